# Initial kernel scaffold; baseline (speedup 1.0000x reference)
import functools

import jax
import jax.numpy as jnp
from jax import lax
from jax.experimental import pallas as pl
from jax.experimental.pallas import tpu as pltpu
from jax.experimental.pallas import tpu_sc as plsc

NC = 2
NS = 16
NW = NC * NS
LANES = 16
G = 128


def _mesh():
    return plsc.VectorSubcoreMesh(core_axis_name="c", subcore_axis_name="s")


def _cdiv(a, b):
    return (a + b - 1) // b


def _sc_degree(n, ng):
    win = _cdiv(ng, NW)
    rpt = n // NS

    @functools.partial(
        pl.kernel,
        out_type=jax.ShapeDtypeStruct((NC, n, LANES), jnp.float32),
        mesh=_mesh(),
        scratch_types=[
            pltpu.VMEM((win, G), jnp.int32),
            pltpu.VMEM((G, LANES), jnp.float32),
            pltpu.VMEM((G, LANES), jnp.float32),
            pltpu.VMEM_SHARED((n, LANES), jnp.float32),
        ],
    )
    def deg_kernel(dst_hbm, out_hbm, idx_v, ones_v, buf_v, acc_sh):
        cid = lax.axis_index("c")
        sid = lax.axis_index("s")
        wid = sid * NC + cid

        def initrow(i, carry):
            ones_v[i, :] = jnp.full((LANES,), 1.0, jnp.float32)
            buf_v[i, :] = jnp.zeros((LANES,), jnp.float32)
            return carry

        lax.fori_loop(0, G, initrow, 0)

        base = sid * rpt
        nfull = rpt // G
        rem = rpt - nfull * G
        for k in range(nfull):
            pltpu.sync_copy(buf_v, acc_sh.at[pl.ds(base + k * G, G)])
        if rem:
            pltpu.sync_copy(buf_v.at[pl.ds(0, rem)],
                            acc_sh.at[pl.ds(base + nfull * G, rem)])
        plsc.subcore_barrier()

        c0 = (ng * wid) // NW
        c1 = (ng * (wid + 1)) // NW
        pltpu.sync_copy(dst_hbm.at[pl.ds(c0, win)], idx_v)

        def edge_group(j, carry):
            pltpu.sync_copy(ones_v, acc_sh.at[idx_v.at[j]], add=True)
            return carry

        lax.fori_loop(0, c1 - c0, edge_group, 0)
        plsc.subcore_barrier()

        for k in range(nfull):
            pltpu.sync_copy(acc_sh.at[pl.ds(base + k * G, G)], buf_v)
            pltpu.sync_copy(buf_v, out_hbm.at[cid, pl.ds(base + k * G, G)])
        if rem:
            pltpu.sync_copy(acc_sh.at[pl.ds(base + nfull * G, rem)],
                            buf_v.at[pl.ds(0, rem)])
            pltpu.sync_copy(buf_v.at[pl.ds(0, rem)],
                            out_hbm.at[cid, pl.ds(base + nfull * G, rem)])

    return deg_kernel


def _sc_agg(n, d, ng):
    win = _cdiv(ng, NW)
    rpt = n // NS

    @functools.partial(
        pl.kernel,
        out_type=jax.ShapeDtypeStruct((NC, n, d), jnp.float32),
        mesh=_mesh(),
        scratch_types=[
            pltpu.VMEM((win, G), jnp.int32),
            pltpu.VMEM((win, G), jnp.int32),
            pltpu.VMEM((G, d), jnp.float32),
            pltpu.VMEM((G, d), jnp.float32),
            pltpu.SemaphoreType.DMA,
            pltpu.VMEM_SHARED((n, d), jnp.float32),
        ],
    )
    def agg_kernel(y_hbm, src_hbm, dst_hbm, out_hbm,
                   idxs_v, idxd_v, rows_v, buf_v, sem, acc_sh):
        cid = lax.axis_index("c")
        sid = lax.axis_index("s")
        wid = sid * NC + cid

        def zrow(i, carry):
            for c in range(d // LANES):
                buf_v[i, pl.ds(c * LANES, LANES)] = jnp.zeros(
                    (LANES,), jnp.float32)
            return carry

        lax.fori_loop(0, G, zrow, 0)

        base = sid * rpt
        nfull = rpt // G
        rem = rpt - nfull * G
        for k in range(nfull):
            pltpu.sync_copy(buf_v, acc_sh.at[pl.ds(base + k * G, G)])
        if rem:
            pltpu.sync_copy(buf_v.at[pl.ds(0, rem)],
                            acc_sh.at[pl.ds(base + nfull * G, rem)])
        plsc.subcore_barrier()

        c0 = (ng * wid) // NW
        c1 = (ng * (wid + 1)) // NW
        pltpu.sync_copy(src_hbm.at[pl.ds(c0, win)], idxs_v)
        pltpu.sync_copy(dst_hbm.at[pl.ds(c0, win)], idxd_v)

        def edge_group(j, carry):
            pltpu.async_copy(y_hbm.at[idxs_v.at[j]], rows_v, sem).wait()
            pltpu.sync_copy(rows_v, acc_sh.at[idxd_v.at[j]], add=True)
            return carry

        lax.fori_loop(0, c1 - c0, edge_group, 0)
        plsc.subcore_barrier()

        for k in range(nfull):
            pltpu.sync_copy(acc_sh.at[pl.ds(base + k * G, G)], buf_v)
            pltpu.sync_copy(buf_v, out_hbm.at[cid, pl.ds(base + k * G, G)])
        if rem:
            pltpu.sync_copy(acc_sh.at[pl.ds(base + nfull * G, rem)],
                            buf_v.at[pl.ds(0, rem)])
            pltpu.sync_copy(buf_v.at[pl.ds(0, rem)],
                            out_hbm.at[cid, pl.ds(base + nfull * G, rem)])

    return agg_kernel


def _dinv_from_partials(p_ref):
    deg = p_ref[0][:, :1] + p_ref[1][:, :1] + 1.0
    return lax.rsqrt(deg)


def _tc_scale(x, degp):
    n, din = x.shape
    blk = 2000

    def body(x_ref, p_ref, y_ref):
        dinv = _dinv_from_partials(p_ref)
        y_ref[...] = x_ref[...] * dinv

    return pl.pallas_call(
        body,
        grid=(n // blk,),
        in_specs=[
            pl.BlockSpec((blk, din), lambda i: (i, 0)),
            pl.BlockSpec((NC, blk, LANES), lambda i: (0, i, 0)),
        ],
        out_specs=pl.BlockSpec((blk, din), lambda i: (i, 0)),
        out_shape=jax.ShapeDtypeStruct((n, din), jnp.float32),
    )(x, degp)


def _tc_mlp(aggp, y, degp, W1, b1, W2):
    n, din = y.shape
    dhid = W1.shape[1]
    dout = W2.shape[1]
    blk = 1250

    def body(a_ref, y_ref, p_ref, w1_ref, b1_ref, w2_ref, z_ref):
        dinv = _dinv_from_partials(p_ref)
        s = (a_ref[0] + a_ref[1] + y_ref[...]) * dinv
        h = jnp.dot(s, w1_ref[...], preferred_element_type=jnp.float32)
        h = jnp.maximum(h + b1_ref[...], 0.0)
        z = jnp.dot(h, w2_ref[...], preferred_element_type=jnp.float32)
        z_ref[...] = z * dinv

    return pl.pallas_call(
        body,
        grid=(n // blk,),
        in_specs=[
            pl.BlockSpec((NC, blk, din), lambda i: (0, i, 0)),
            pl.BlockSpec((blk, din), lambda i: (i, 0)),
            pl.BlockSpec((NC, blk, LANES), lambda i: (0, i, 0)),
            pl.BlockSpec((din, dhid), lambda i: (0, 0)),
            pl.BlockSpec((1, dhid), lambda i: (0, 0)),
            pl.BlockSpec((dhid, dout), lambda i: (0, 0)),
        ],
        out_specs=pl.BlockSpec((blk, dout), lambda i: (i, 0)),
        out_shape=jax.ShapeDtypeStruct((n, dout), jnp.float32),
    )(aggp, y, degp, W1, b1.reshape(1, dhid), W2)


def _tc_final(cp, z, degp, b2):
    n, dout = z.shape
    blk = 2500

    def body(c_ref, z_ref, p_ref, b2_ref, o_ref):
        dinv = _dinv_from_partials(p_ref)
        o = (c_ref[0] + c_ref[1] + z_ref[...]) * dinv + b2_ref[...]
        m = jnp.max(o, axis=1, keepdims=True)
        e = jnp.exp(o - m)
        s = jnp.sum(e, axis=1, keepdims=True)
        o_ref[...] = (o - m) - jnp.log(s)

    return pl.pallas_call(
        body,
        grid=(n // blk,),
        in_specs=[
            pl.BlockSpec((NC, blk, dout), lambda i: (0, i, 0)),
            pl.BlockSpec((blk, dout), lambda i: (i, 0)),
            pl.BlockSpec((NC, blk, LANES), lambda i: (0, i, 0)),
            pl.BlockSpec((1, dout), lambda i: (0, 0)),
        ],
        out_specs=pl.BlockSpec((blk, dout), lambda i: (i, 0)),
        out_shape=jax.ShapeDtypeStruct((n, dout), jnp.float32),
    )(cp, z, degp, b2.reshape(1, dout))


def kernel(x, edge_index, W1, b1, W2, b2):
    n, din = x.shape
    e = edge_index.shape[1]
    ng = e // G
    src2d = edge_index[0].reshape(ng, G)
    dst2d = edge_index[1].reshape(ng, G)

    degp = _sc_degree(n, ng)(dst2d)
    y = _tc_scale(x, degp)
    aggp = _sc_agg(n, din, ng)(y, src2d, dst2d)
    z = _tc_mlp(aggp, y, degp, W1, b1, W2)
    cp = _sc_agg(n, W2.shape[1], ng)(z, src2d, dst2d)
    return _tc_final(cp, z, degp, b2)


# speedup vs baseline: 20.8038x; 20.8038x over previous
import functools

import jax
import jax.numpy as jnp
from jax import lax
from jax.experimental import pallas as pl
from jax.experimental.pallas import tpu as pltpu
from jax.experimental.pallas import tpu_sc as plsc

NC = 2
NS = 16
NW = NC * NS
LANES = 16
G = 64


def _mesh():
    return plsc.VectorSubcoreMesh(core_axis_name="c", subcore_axis_name="s")


def _cdiv(a, b):
    return (a + b - 1) // b


def _win_size(ng):
    nb = ng // 8
    return 8 * _cdiv(nb, NW) + (ng - 8 * nb)


def _group_range(wid, ng):
    nb = ng // 8
    rg = ng - 8 * nb
    b0 = (nb * wid) // NW
    b1 = (nb * (wid + 1)) // NW
    c0 = 8 * b0
    nj = 8 * (b1 - b0) + jnp.where(wid == NW - 1, rg, 0)
    return c0, nj


def _row_window(n):
    return 8 * _cdiv(n // 8, NS)


def _row_start(sid, n):
    return 8 * (((n // 8) * sid) // NS)


def _sc_degree(n, ng):
    win = _win_size(ng)
    wr = _row_window(n)

    @functools.partial(
        pl.kernel,
        out_type=jax.ShapeDtypeStruct((NC, n, LANES), jnp.float32),
        mesh=_mesh(),
        scratch_types=[
            pltpu.VMEM((win, G), jnp.int32),
            pltpu.VMEM((G, LANES), jnp.float32),
            pltpu.VMEM((G, LANES), jnp.float32),
            pltpu.VMEM_SHARED((n, LANES), jnp.float32),
        ],
        compiler_params=pltpu.CompilerParams(use_tc_tiling_on_sc=False),
    )
    def deg_kernel(dst_hbm, out_hbm, idx_v, ones_v, buf_v, acc_sh):
        cid = lax.axis_index("c")
        sid = lax.axis_index("s")
        wid = sid * NC + cid

        def initrow(i, carry):
            ones_v[i, :] = jnp.full((LANES,), 1.0, jnp.float32)
            buf_v[i, :] = jnp.zeros((LANES,), jnp.float32)
            return carry

        lax.fori_loop(0, G, initrow, 0)

        base = _row_start(sid, n)
        nfull = wr // G
        rem = wr - nfull * G
        for k in range(nfull):
            pltpu.sync_copy(buf_v, acc_sh.at[pl.ds(base + k * G, G)])
        if rem:
            pltpu.sync_copy(buf_v.at[pl.ds(0, rem)],
                            acc_sh.at[pl.ds(base + nfull * G, rem)])
        plsc.subcore_barrier()

        c0, nj = _group_range(wid, ng)
        pltpu.sync_copy(dst_hbm.at[pl.ds(c0, win)], idx_v)

        def edge_group(j, carry):
            pltpu.sync_copy(ones_v, acc_sh.at[idx_v.at[j]], add=True)
            return carry

        lax.fori_loop(0, nj, edge_group, 0)
        plsc.subcore_barrier()

        for k in range(nfull):
            pltpu.sync_copy(acc_sh.at[pl.ds(base + k * G, G)], buf_v)
            pltpu.sync_copy(buf_v, out_hbm.at[cid, pl.ds(base + k * G, G)])
        if rem:
            pltpu.sync_copy(acc_sh.at[pl.ds(base + nfull * G, rem)],
                            buf_v.at[pl.ds(0, rem)])
            pltpu.sync_copy(buf_v.at[pl.ds(0, rem)],
                            out_hbm.at[cid, pl.ds(base + nfull * G, rem)])

    return deg_kernel


def _sc_agg_cols(n, dh, ng):
    nb = ng // 8
    rg = ng - 8 * nb
    win = 8 * _cdiv(nb, NS) + rg
    wr = _row_window(n)

    @functools.partial(
        pl.kernel,
        out_type=jax.ShapeDtypeStruct((NC, n, dh), jnp.float32),
        mesh=_mesh(),
        scratch_types=[
            pltpu.VMEM((win, G), jnp.int32),
            pltpu.VMEM((win, G), jnp.int32),
            pltpu.VMEM((G, dh), jnp.float32),
            pltpu.VMEM((G, dh), jnp.float32),
            pltpu.SemaphoreType.DMA,
            pltpu.VMEM_SHARED((n, dh), jnp.float32),
        ],
        compiler_params=pltpu.CompilerParams(use_tc_tiling_on_sc=False),
    )
    def agg_kernel(y_hbm, src_hbm, dst_hbm, out_hbm,
                   idxs_v, idxd_v, rows_v, buf_v, sem, acc_sh):
        cid = lax.axis_index("c")
        sid = lax.axis_index("s")

        def zrow(i, carry):
            for c in range(dh // LANES):
                buf_v[i, pl.ds(c * LANES, LANES)] = jnp.zeros(
                    (LANES,), jnp.float32)
            return carry

        lax.fori_loop(0, G, zrow, 0)

        base = _row_start(sid, n)
        nfull = wr // G
        rem = wr - nfull * G
        for k in range(nfull):
            pltpu.sync_copy(buf_v, acc_sh.at[pl.ds(base + k * G, G)])
        if rem:
            pltpu.sync_copy(buf_v.at[pl.ds(0, rem)],
                            acc_sh.at[pl.ds(base + nfull * G, rem)])
        plsc.subcore_barrier()

        b0 = (nb * sid) // NS
        b1 = (nb * (sid + 1)) // NS
        c0 = 8 * b0
        nj = 8 * (b1 - b0) + jnp.where(sid == NS - 1, rg, 0)
        pltpu.sync_copy(src_hbm.at[pl.ds(c0, win)], idxs_v)
        pltpu.sync_copy(dst_hbm.at[pl.ds(c0, win)], idxd_v)

        def make_edge_group(y_half):
            def edge_group(j, carry):
                pltpu.async_copy(y_half.at[idxs_v.at[j]], rows_v, sem).wait()
                pltpu.sync_copy(rows_v, acc_sh.at[idxd_v.at[j]], add=True)
                return carry
            return edge_group

        @pl.when(cid == 0)
        def _():
            lax.fori_loop(0, nj, make_edge_group(y_hbm.at[0]), 0)

        @pl.when(cid == 1)
        def _():
            lax.fori_loop(0, nj, make_edge_group(y_hbm.at[1]), 0)

        plsc.subcore_barrier()

        for k in range(nfull):
            pltpu.sync_copy(acc_sh.at[pl.ds(base + k * G, G)], buf_v)
            pltpu.sync_copy(buf_v, out_hbm.at[cid, pl.ds(base + k * G, G)])
        if rem:
            pltpu.sync_copy(acc_sh.at[pl.ds(base + nfull * G, rem)],
                            buf_v.at[pl.ds(0, rem)])
            pltpu.sync_copy(buf_v.at[pl.ds(0, rem)],
                            out_hbm.at[cid, pl.ds(base + nfull * G, rem)])

    return agg_kernel


def _sc_agg(n, d, ng):
    win = _win_size(ng)
    wr = _row_window(n)

    @functools.partial(
        pl.kernel,
        out_type=jax.ShapeDtypeStruct((NC, n, d), jnp.float32),
        mesh=_mesh(),
        scratch_types=[
            pltpu.VMEM((win, G), jnp.int32),
            pltpu.VMEM((win, G), jnp.int32),
            pltpu.VMEM((G, d), jnp.float32),
            pltpu.VMEM((G, d), jnp.float32),
            pltpu.SemaphoreType.DMA,
            pltpu.VMEM_SHARED((n, d), jnp.float32),
        ],
        compiler_params=pltpu.CompilerParams(use_tc_tiling_on_sc=False),
    )
    def agg_kernel(y_hbm, src_hbm, dst_hbm, out_hbm,
                   idxs_v, idxd_v, rows_v, buf_v, sem, acc_sh):
        cid = lax.axis_index("c")
        sid = lax.axis_index("s")
        wid = sid * NC + cid

        def zrow(i, carry):
            for c in range(d // LANES):
                buf_v[i, pl.ds(c * LANES, LANES)] = jnp.zeros(
                    (LANES,), jnp.float32)
            return carry

        lax.fori_loop(0, G, zrow, 0)

        base = _row_start(sid, n)
        nfull = wr // G
        rem = wr - nfull * G
        for k in range(nfull):
            pltpu.sync_copy(buf_v, acc_sh.at[pl.ds(base + k * G, G)])
        if rem:
            pltpu.sync_copy(buf_v.at[pl.ds(0, rem)],
                            acc_sh.at[pl.ds(base + nfull * G, rem)])
        plsc.subcore_barrier()

        c0, nj = _group_range(wid, ng)
        pltpu.sync_copy(src_hbm.at[pl.ds(c0, win)], idxs_v)
        pltpu.sync_copy(dst_hbm.at[pl.ds(c0, win)], idxd_v)

        def edge_group(j, carry):
            pltpu.async_copy(y_hbm.at[idxs_v.at[j]], rows_v, sem).wait()
            pltpu.sync_copy(rows_v, acc_sh.at[idxd_v.at[j]], add=True)
            return carry

        lax.fori_loop(0, nj, edge_group, 0)
        plsc.subcore_barrier()

        for k in range(nfull):
            pltpu.sync_copy(acc_sh.at[pl.ds(base + k * G, G)], buf_v)
            pltpu.sync_copy(buf_v, out_hbm.at[cid, pl.ds(base + k * G, G)])
        if rem:
            pltpu.sync_copy(acc_sh.at[pl.ds(base + nfull * G, rem)],
                            buf_v.at[pl.ds(0, rem)])
            pltpu.sync_copy(buf_v.at[pl.ds(0, rem)],
                            out_hbm.at[cid, pl.ds(base + nfull * G, rem)])

    return agg_kernel


def _dinv_from_partials(p_ref):
    deg = p_ref[0][:, :1] + p_ref[1][:, :1] + 1.0
    return lax.rsqrt(deg)


def _tc_scale(x, degp):
    n, din = x.shape
    dh = din // NC
    blk = 2000

    def body(x_ref, p_ref, y_ref):
        dinv = _dinv_from_partials(p_ref)
        y_ref[0] = x_ref[:, :dh] * dinv
        y_ref[1] = x_ref[:, dh:] * dinv

    return pl.pallas_call(
        body,
        grid=(n // blk,),
        in_specs=[
            pl.BlockSpec((blk, din), lambda i: (i, 0)),
            pl.BlockSpec((NC, blk, LANES), lambda i: (0, i, 0)),
        ],
        out_specs=pl.BlockSpec((NC, blk, dh), lambda i: (0, i, 0)),
        out_shape=jax.ShapeDtypeStruct((NC, n, dh), jnp.float32),
    )(x, degp)


def _tc_mlp(aggp, y2, degp, W1, b1, W2):
    nc, n, dh = y2.shape
    din = nc * dh
    dhid = W1.shape[1]
    dout = W2.shape[1]
    blk = 1000

    def body(a_ref, y_ref, p_ref, w1_ref, b1_ref, w2_ref, z_ref):
        dinv = _dinv_from_partials(p_ref)
        s0 = (a_ref[0] + y_ref[0]) * dinv
        s1 = (a_ref[1] + y_ref[1]) * dinv
        s = jnp.concatenate([s0, s1], axis=1)
        h = jnp.dot(s, w1_ref[...], preferred_element_type=jnp.float32)
        h = jnp.maximum(h + b1_ref[...], 0.0)
        z = jnp.dot(h, w2_ref[...], preferred_element_type=jnp.float32)
        z_ref[...] = z * dinv

    return pl.pallas_call(
        body,
        grid=(n // blk,),
        in_specs=[
            pl.BlockSpec((NC, blk, dh), lambda i: (0, i, 0)),
            pl.BlockSpec((NC, blk, dh), lambda i: (0, i, 0)),
            pl.BlockSpec((NC, blk, LANES), lambda i: (0, i, 0)),
            pl.BlockSpec((din, dhid), lambda i: (0, 0)),
            pl.BlockSpec((1, dhid), lambda i: (0, 0)),
            pl.BlockSpec((dhid, dout), lambda i: (0, 0)),
        ],
        out_specs=pl.BlockSpec((blk, dout), lambda i: (i, 0)),
        out_shape=jax.ShapeDtypeStruct((n, dout), jnp.float32),
    )(aggp, y2, degp, W1, b1.reshape(1, dhid), W2)


def _tc_final(cp, z, degp, b2):
    n, dout = z.shape
    blk = 2000

    def body(c_ref, z_ref, p_ref, b2_ref, o_ref):
        dinv = _dinv_from_partials(p_ref)
        o = (c_ref[0] + c_ref[1] + z_ref[...]) * dinv + b2_ref[...]
        m = jnp.max(o, axis=1, keepdims=True)
        e = jnp.exp(o - m)
        s = jnp.sum(e, axis=1, keepdims=True)
        o_ref[...] = (o - m) - jnp.log(s)

    return pl.pallas_call(
        body,
        grid=(n // blk,),
        in_specs=[
            pl.BlockSpec((NC, blk, dout), lambda i: (0, i, 0)),
            pl.BlockSpec((blk, dout), lambda i: (i, 0)),
            pl.BlockSpec((NC, blk, LANES), lambda i: (0, i, 0)),
            pl.BlockSpec((1, dout), lambda i: (0, 0)),
        ],
        out_specs=pl.BlockSpec((blk, dout), lambda i: (i, 0)),
        out_shape=jax.ShapeDtypeStruct((n, dout), jnp.float32),
    )(cp, z, degp, b2.reshape(1, dout))


def kernel(x, edge_index, W1, b1, W2, b2):
    n, din = x.shape
    e = edge_index.shape[1]
    ng = e // G
    src2d = edge_index[0].reshape(ng, G)
    dst2d = edge_index[1].reshape(ng, G)

    degp = _sc_degree(n, ng)(dst2d)
    y2 = _tc_scale(x, degp)
    aggp = _sc_agg_cols(n, din // NC, ng)(y2, src2d, dst2d)
    z = _tc_mlp(aggp, y2, degp, W1, b1, W2)
    cp = _sc_agg(n, W2.shape[1], ng)(z, src2d, dst2d)
    return _tc_final(cp, z, degp, b2)


# speedup vs baseline: 46.2943x; 2.2253x over previous
import functools

import jax
import jax.numpy as jnp
from jax import lax
from jax.experimental import pallas as pl
from jax.experimental.pallas import tpu as pltpu
from jax.experimental.pallas import tpu_sc as plsc

NC = 2
NS = 16
NW = NC * NS
LANES = 16
G = 64


def _mesh():
    return plsc.VectorSubcoreMesh(core_axis_name="c", subcore_axis_name="s")


def _cdiv(a, b):
    return (a + b - 1) // b


def _win_size(ng):
    nb = ng // 8
    return 8 * _cdiv(nb, NW) + (ng - 8 * nb)


def _group_range(wid, ng):
    nb = ng // 8
    rg = ng - 8 * nb
    b0 = (nb * wid) // NW
    b1 = (nb * (wid + 1)) // NW
    c0 = 8 * b0
    nj = 8 * (b1 - b0) + jnp.where(wid == NW - 1, rg, 0)
    return c0, nj


def _row_window(n):
    return 8 * _cdiv(n // 8, NS)


def _row_start(sid, n):
    return 8 * (((n // 8) * sid) // NS)


def _sc_degree(n, ng):
    win = _win_size(ng)
    wr = _row_window(n)

    @functools.partial(
        pl.kernel,
        out_type=jax.ShapeDtypeStruct((NC, n, LANES), jnp.float32),
        mesh=_mesh(),
        scratch_types=[
            pltpu.VMEM((win, G), jnp.int32),
            pltpu.VMEM((G, LANES), jnp.float32),
            pltpu.VMEM((G, LANES), jnp.float32),
            pltpu.SemaphoreType.DMA,
            pltpu.VMEM_SHARED((n, LANES), jnp.float32),
        ],
        compiler_params=pltpu.CompilerParams(use_tc_tiling_on_sc=False),
    )
    def deg_kernel(dst_hbm, out_hbm, idx_v, ones_v, buf_v, sem, acc_sh):
        cid = lax.axis_index("c")
        sid = lax.axis_index("s")
        wid = sid * NC + cid

        def initrow(i, carry):
            ones_v[i, :] = jnp.full((LANES,), 1.0, jnp.float32)
            buf_v[i, :] = jnp.zeros((LANES,), jnp.float32)
            return carry

        lax.fori_loop(0, G, initrow, 0)

        base = _row_start(sid, n)
        nfull = wr // G
        rem = wr - nfull * G
        for k in range(nfull):
            pltpu.sync_copy(buf_v, acc_sh.at[pl.ds(base + k * G, G)])
        if rem:
            pltpu.sync_copy(buf_v.at[pl.ds(0, rem)],
                            acc_sh.at[pl.ds(base + nfull * G, rem)])
        plsc.subcore_barrier()

        c0, nj = _group_range(wid, ng)
        pltpu.sync_copy(dst_hbm.at[pl.ds(c0, win)], idx_v)

        def edge_group(j, carry):
            pltpu.async_copy(ones_v, acc_sh.at[idx_v.at[j]], sem, add=True)
            return carry

        lax.fori_loop(0, nj, edge_group, 0)

        def edge_drain(j, carry):
            pltpu.make_async_copy(ones_v, acc_sh.at[idx_v.at[j]], sem).wait()
            return carry

        lax.fori_loop(0, nj, edge_drain, 0)
        plsc.subcore_barrier()

        for k in range(nfull):
            pltpu.sync_copy(acc_sh.at[pl.ds(base + k * G, G)], buf_v)
            pltpu.sync_copy(buf_v, out_hbm.at[cid, pl.ds(base + k * G, G)])
        if rem:
            pltpu.sync_copy(acc_sh.at[pl.ds(base + nfull * G, rem)],
                            buf_v.at[pl.ds(0, rem)])
            pltpu.sync_copy(buf_v.at[pl.ds(0, rem)],
                            out_hbm.at[cid, pl.ds(base + nfull * G, rem)])

    return deg_kernel


NBUF = 8


def _build_zero_rows(buf, d):
    def zrow(i, carry):
        for c in range(d // LANES):
            buf[i, pl.ds(c * LANES, LANES)] = jnp.zeros((LANES,), jnp.float32)
        return carry
    lax.fori_loop(0, G, zrow, 0)


def _zero_acc(acc_sh, zbuf, sem, base, wr):
    nfz = wr // G
    rz = wr - nfz * G
    for k in range(nfz):
        pltpu.async_copy(zbuf, acc_sh.at[pl.ds(base + k * G, G)], sem)
    if rz:
        pltpu.async_copy(zbuf.at[pl.ds(0, rz)],
                         acc_sh.at[pl.ds(base + nfz * G, rz)], sem)
    for k in range(nfz):
        pltpu.make_async_copy(zbuf, acc_sh.at[pl.ds(base + k * G, G)],
                              sem).wait()
    if rz:
        pltpu.make_async_copy(zbuf.at[pl.ds(0, rz)],
                              acc_sh.at[pl.ds(base + nfz * G, rz)],
                              sem).wait()


def _edge_pipeline(y_src, idxs_v, idxd_v, rows_v, gsems, ssems, acc_sh, nj):
    def outer(g, carry):
        j0 = g * NBUF
        for b in range(NBUF):
            j = j0 + b

            @pl.when(jnp.logical_and(j < nj, j >= NBUF))
            def _(b=b, j=j):
                pltpu.make_async_copy(rows_v.at[b],
                                      acc_sh.at[idxd_v.at[j - NBUF]],
                                      ssems[b]).wait()

            @pl.when(j < nj)
            def _(b=b, j=j):
                pltpu.async_copy(y_src.at[idxs_v.at[j]], rows_v.at[b],
                                 gsems[b])
        for b in range(NBUF):
            j = j0 + b

            @pl.when(j < nj)
            def _(b=b, j=j):
                pltpu.make_async_copy(y_src.at[idxs_v.at[j]], rows_v.at[b],
                                      gsems[b]).wait()
                pltpu.async_copy(rows_v.at[b], acc_sh.at[idxd_v.at[j]],
                                 ssems[b], add=True)
        return carry

    lax.fori_loop(0, (nj + NBUF - 1) // NBUF, outer, 0)
    for b in range(NBUF):
        @pl.when(b < nj)
        def _(b=b):
            pltpu.make_async_copy(rows_v.at[b], acc_sh.at[idxd_v.at[0]],
                                  ssems[b]).wait()


def _copy_out(acc_sh, rows_v, ssems, out_hbm, cid, base, wr):
    nfull = wr // G
    rem = wr - nfull * G
    nchunks = nfull + (1 if rem else 0)

    def chunk(k):
        if k < nfull:
            return base + k * G, G
        return base + nfull * G, rem

    def bufref(b, sz):
        return rows_v.at[b] if sz == G else rows_v.at[b, pl.ds(0, sz)]

    for k in range(nchunks):
        b = k % NBUF
        off, sz = chunk(k)
        if k >= NBUF:
            poff, psz = chunk(k - NBUF)
            pltpu.make_async_copy(bufref(b, psz),
                                  out_hbm.at[cid, pl.ds(poff, psz)],
                                  ssems[b]).wait()
        pltpu.sync_copy(acc_sh.at[pl.ds(off, sz)], bufref(b, sz))
        pltpu.async_copy(bufref(b, sz), out_hbm.at[cid, pl.ds(off, sz)],
                         ssems[b])
    for k in range(max(0, nchunks - NBUF), nchunks):
        b = k % NBUF
        off, sz = chunk(k)
        pltpu.make_async_copy(bufref(b, sz), out_hbm.at[cid, pl.ds(off, sz)],
                              ssems[b]).wait()


def _sc_agg_cols(n, dh, ng):
    nb = ng // 8
    rg = ng - 8 * nb
    win = 8 * _cdiv(nb, NS) + rg
    wr = _row_window(n)

    @functools.partial(
        pl.kernel,
        out_type=jax.ShapeDtypeStruct((NC, n, dh), jnp.float32),
        mesh=_mesh(),
        scratch_types=(
            [pltpu.VMEM((win, G), jnp.int32),
             pltpu.VMEM((win, G), jnp.int32),
             pltpu.VMEM((NBUF, G, dh), jnp.float32)]
            + [pltpu.SemaphoreType.DMA] * (2 * NBUF)
            + [pltpu.VMEM_SHARED((n, dh), jnp.float32)]
        ),
        compiler_params=pltpu.CompilerParams(use_tc_tiling_on_sc=False),
    )
    def agg_kernel(y_hbm, src_hbm, dst_hbm, out_hbm,
                   idxs_v, idxd_v, rows_v, *rest):
        gsems = rest[:NBUF]
        ssems = rest[NBUF:2 * NBUF]
        acc_sh = rest[2 * NBUF]
        cid = lax.axis_index("c")
        sid = lax.axis_index("s")

        _build_zero_rows(rows_v.at[0], dh)
        base = _row_start(sid, n)
        _zero_acc(acc_sh, rows_v.at[0], gsems[0], base, wr)
        plsc.subcore_barrier()

        b0 = (nb * sid) // NS
        b1 = (nb * (sid + 1)) // NS
        c0 = 8 * b0
        nj = 8 * (b1 - b0) + jnp.where(sid == NS - 1, rg, 0)
        pltpu.sync_copy(src_hbm.at[pl.ds(c0, win)], idxs_v)
        pltpu.sync_copy(dst_hbm.at[pl.ds(c0, win)], idxd_v)

        @pl.when(cid == 0)
        def _():
            _edge_pipeline(y_hbm.at[0], idxs_v, idxd_v, rows_v,
                           gsems, ssems, acc_sh, nj)

        @pl.when(cid == 1)
        def _():
            _edge_pipeline(y_hbm.at[1], idxs_v, idxd_v, rows_v,
                           gsems, ssems, acc_sh, nj)

        plsc.subcore_barrier()
        _copy_out(acc_sh, rows_v, ssems, out_hbm, cid, base, wr)

    return agg_kernel


def _sc_agg(n, d, ng):
    win = _win_size(ng)
    wr = _row_window(n)

    @functools.partial(
        pl.kernel,
        out_type=jax.ShapeDtypeStruct((NC, n, d), jnp.float32),
        mesh=_mesh(),
        scratch_types=(
            [pltpu.VMEM((win, G), jnp.int32),
             pltpu.VMEM((win, G), jnp.int32),
             pltpu.VMEM((NBUF, G, d), jnp.float32)]
            + [pltpu.SemaphoreType.DMA] * (2 * NBUF)
            + [pltpu.VMEM_SHARED((n, d), jnp.float32)]
        ),
        compiler_params=pltpu.CompilerParams(use_tc_tiling_on_sc=False),
    )
    def agg_kernel(y_hbm, src_hbm, dst_hbm, out_hbm,
                   idxs_v, idxd_v, rows_v, *rest):
        gsems = rest[:NBUF]
        ssems = rest[NBUF:2 * NBUF]
        acc_sh = rest[2 * NBUF]
        cid = lax.axis_index("c")
        sid = lax.axis_index("s")
        wid = sid * NC + cid

        _build_zero_rows(rows_v.at[0], d)
        base = _row_start(sid, n)
        _zero_acc(acc_sh, rows_v.at[0], gsems[0], base, wr)
        plsc.subcore_barrier()

        c0, nj = _group_range(wid, ng)
        pltpu.sync_copy(src_hbm.at[pl.ds(c0, win)], idxs_v)
        pltpu.sync_copy(dst_hbm.at[pl.ds(c0, win)], idxd_v)

        _edge_pipeline(y_hbm, idxs_v, idxd_v, rows_v,
                       gsems, ssems, acc_sh, nj)

        plsc.subcore_barrier()
        _copy_out(acc_sh, rows_v, ssems, out_hbm, cid, base, wr)

    return agg_kernel


def _dinv_from_partials(p_ref):
    deg = p_ref[0][:, :1] + p_ref[1][:, :1] + 1.0
    return lax.rsqrt(deg)


def _tc_scale(x, degp):
    n, din = x.shape
    dh = din // NC
    blk = 2000

    def body(x_ref, p_ref, y_ref):
        dinv = _dinv_from_partials(p_ref)
        y_ref[0] = x_ref[:, :dh] * dinv
        y_ref[1] = x_ref[:, dh:] * dinv

    return pl.pallas_call(
        body,
        grid=(n // blk,),
        in_specs=[
            pl.BlockSpec((blk, din), lambda i: (i, 0)),
            pl.BlockSpec((NC, blk, LANES), lambda i: (0, i, 0)),
        ],
        out_specs=pl.BlockSpec((NC, blk, dh), lambda i: (0, i, 0)),
        out_shape=jax.ShapeDtypeStruct((NC, n, dh), jnp.float32),
    )(x, degp)


def _tc_mlp(aggp, y2, degp, W1, b1, W2):
    nc, n, dh = y2.shape
    din = nc * dh
    dhid = W1.shape[1]
    dout = W2.shape[1]
    blk = 1000

    def body(a_ref, y_ref, p_ref, w1_ref, b1_ref, w2_ref, z_ref):
        dinv = _dinv_from_partials(p_ref)
        s0 = (a_ref[0] + y_ref[0]) * dinv
        s1 = (a_ref[1] + y_ref[1]) * dinv
        s = jnp.concatenate([s0, s1], axis=1)
        h = jnp.dot(s, w1_ref[...], preferred_element_type=jnp.float32)
        h = jnp.maximum(h + b1_ref[...], 0.0)
        z = jnp.dot(h, w2_ref[...], preferred_element_type=jnp.float32)
        z_ref[...] = z * dinv

    return pl.pallas_call(
        body,
        grid=(n // blk,),
        in_specs=[
            pl.BlockSpec((NC, blk, dh), lambda i: (0, i, 0)),
            pl.BlockSpec((NC, blk, dh), lambda i: (0, i, 0)),
            pl.BlockSpec((NC, blk, LANES), lambda i: (0, i, 0)),
            pl.BlockSpec((din, dhid), lambda i: (0, 0)),
            pl.BlockSpec((1, dhid), lambda i: (0, 0)),
            pl.BlockSpec((dhid, dout), lambda i: (0, 0)),
        ],
        out_specs=pl.BlockSpec((blk, dout), lambda i: (i, 0)),
        out_shape=jax.ShapeDtypeStruct((n, dout), jnp.float32),
    )(aggp, y2, degp, W1, b1.reshape(1, dhid), W2)


def _tc_final(cp, z, degp, b2):
    n, dout = z.shape
    blk = 2000

    def body(c_ref, z_ref, p_ref, b2_ref, o_ref):
        dinv = _dinv_from_partials(p_ref)
        o = (c_ref[0] + c_ref[1] + z_ref[...]) * dinv + b2_ref[...]
        m = jnp.max(o, axis=1, keepdims=True)
        e = jnp.exp(o - m)
        s = jnp.sum(e, axis=1, keepdims=True)
        o_ref[...] = (o - m) - jnp.log(s)

    return pl.pallas_call(
        body,
        grid=(n // blk,),
        in_specs=[
            pl.BlockSpec((NC, blk, dout), lambda i: (0, i, 0)),
            pl.BlockSpec((blk, dout), lambda i: (i, 0)),
            pl.BlockSpec((NC, blk, LANES), lambda i: (0, i, 0)),
            pl.BlockSpec((1, dout), lambda i: (0, 0)),
        ],
        out_specs=pl.BlockSpec((blk, dout), lambda i: (i, 0)),
        out_shape=jax.ShapeDtypeStruct((n, dout), jnp.float32),
    )(cp, z, degp, b2.reshape(1, dout))


def kernel(x, edge_index, W1, b1, W2, b2):
    n, din = x.shape
    e = edge_index.shape[1]
    ng = e // G
    src2d = edge_index[0].reshape(ng, G)
    dst2d = edge_index[1].reshape(ng, G)

    degp = _sc_degree(n, ng)(dst2d)
    y2 = _tc_scale(x, degp)
    aggp = _sc_agg_cols(n, din // NC, ng)(y2, src2d, dst2d)
    z = _tc_mlp(aggp, y2, degp, W1, b1, W2)
    cp = _sc_agg(n, W2.shape[1], ng)(z, src2d, dst2d)
    return _tc_final(cp, z, degp, b2)


# speedup vs baseline: 48.7566x; 1.0532x over previous
import functools

import jax
import jax.numpy as jnp
from jax import lax
from jax.experimental import pallas as pl
from jax.experimental.pallas import tpu as pltpu
from jax.experimental.pallas import tpu_sc as plsc

NC = 2
NS = 16
NW = NC * NS
LANES = 16
G = 64


def _mesh():
    return plsc.VectorSubcoreMesh(core_axis_name="c", subcore_axis_name="s")


def _cdiv(a, b):
    return (a + b - 1) // b


def _win_size(ng):
    nb = ng // 8
    return 8 * _cdiv(nb, NW) + (ng - 8 * nb)


def _group_range(wid, ng):
    nb = ng // 8
    rg = ng - 8 * nb
    b0 = (nb * wid) // NW
    b1 = (nb * (wid + 1)) // NW
    c0 = 8 * b0
    nj = 8 * (b1 - b0) + jnp.where(wid == NW - 1, rg, 0)
    return c0, nj


def _row_window(n):
    return 8 * _cdiv(n // 8, NS)


def _row_start(sid, n):
    return 8 * (((n // 8) * sid) // NS)


def _sc_degree(n, ng):
    win = _win_size(ng)
    wr = _row_window(n)

    @functools.partial(
        pl.kernel,
        out_type=jax.ShapeDtypeStruct((NC, n), jnp.float32),
        mesh=_mesh(),
        scratch_types=[
            pltpu.VMEM((win, G), jnp.int32),
            pltpu.VMEM((G, LANES), jnp.float32),
            pltpu.VMEM((G, LANES), jnp.float32),
            pltpu.VMEM((G,), jnp.float32),
            pltpu.SemaphoreType.DMA,
            pltpu.VMEM_SHARED((n, LANES), jnp.float32),
        ],
        compiler_params=pltpu.CompilerParams(use_tc_tiling_on_sc=False,
                                             needs_layout_passes=False),
    )
    def deg_kernel(ei_hbm, out_hbm, idx_v, ones_v, buf_v, col_v, sem, acc_sh):
        cid = lax.axis_index("c")
        sid = lax.axis_index("s")
        wid = sid * NC + cid

        def initrow(i, carry):
            ones_v[i, :] = jnp.full((LANES,), 1.0, jnp.float32)
            buf_v[i, :] = jnp.zeros((LANES,), jnp.float32)
            return carry

        lax.fori_loop(0, G, initrow, 0)

        base = _row_start(sid, n)
        nfull = wr // G
        rem = wr - nfull * G
        for k in range(nfull):
            pltpu.sync_copy(buf_v, acc_sh.at[pl.ds(base + k * G, G)])
        if rem:
            pltpu.sync_copy(buf_v.at[pl.ds(0, rem)],
                            acc_sh.at[pl.ds(base + nfull * G, rem)])
        plsc.subcore_barrier()

        c0, nj = _group_range(wid, ng)
        pltpu.sync_copy(ei_hbm.at[1, pl.ds(c0, win)], idx_v)

        def edge_group(j, carry):
            pltpu.async_copy(ones_v, acc_sh.at[idx_v.at[j]], sem, add=True)
            return carry

        lax.fori_loop(0, nj, edge_group, 0)

        def edge_drain(j, carry):
            pltpu.make_async_copy(ones_v, acc_sh.at[idx_v.at[j]], sem).wait()
            return carry

        lax.fori_loop(0, nj, edge_drain, 0)
        plsc.subcore_barrier()

        lane = lax.iota(jnp.int32, LANES)
        zero16 = jnp.zeros((LANES,), jnp.int32)
        nchunks = nfull + (1 if rem else 0)
        for k in range(nchunks):
            off = base + k * G
            sz = G if k < nfull else rem
            pltpu.sync_copy(acc_sh.at[pl.ds(off, sz)],
                            buf_v if sz == G else buf_v.at[pl.ds(0, sz)])
            for j in range(_cdiv(sz, LANES)):
                rowi = jnp.minimum(j * LANES + lane, sz - 1)
                col_v[pl.ds(j * LANES, LANES)] = plsc.load_gather(
                    buf_v, [rowi, zero16])
            pltpu.sync_copy(col_v if sz == G else col_v.at[pl.ds(0, sz)],
                            out_hbm.at[cid, pl.ds(off, sz)])

    return deg_kernel


NBUF = 8


def _build_zero_rows(buf, d):
    def zrow(i, carry):
        for c in range(d // LANES):
            buf[i, pl.ds(c * LANES, LANES)] = jnp.zeros((LANES,), jnp.float32)
        return carry
    lax.fori_loop(0, G, zrow, 0)


def _zero_acc(acc_sh, zbuf, sem, base, wr):
    nfz = wr // G
    rz = wr - nfz * G
    for k in range(nfz):
        pltpu.async_copy(zbuf, acc_sh.at[pl.ds(base + k * G, G)], sem)
    if rz:
        pltpu.async_copy(zbuf.at[pl.ds(0, rz)],
                         acc_sh.at[pl.ds(base + nfz * G, rz)], sem)
    for k in range(nfz):
        pltpu.make_async_copy(zbuf, acc_sh.at[pl.ds(base + k * G, G)],
                              sem).wait()
    if rz:
        pltpu.make_async_copy(zbuf.at[pl.ds(0, rz)],
                              acc_sh.at[pl.ds(base + nfz * G, rz)],
                              sem).wait()


def _edge_pipeline(y_src, idxs_v, idxd_v, rows_v, gsems, ssems, acc_sh, nj):
    def outer(g, carry):
        j0 = g * NBUF
        for b in range(NBUF):
            j = j0 + b

            @pl.when(jnp.logical_and(j < nj, j >= NBUF))
            def _(b=b, j=j):
                pltpu.make_async_copy(rows_v.at[b],
                                      acc_sh.at[idxd_v.at[j - NBUF]],
                                      ssems[b]).wait()

            @pl.when(j < nj)
            def _(b=b, j=j):
                pltpu.async_copy(y_src.at[idxs_v.at[j]], rows_v.at[b],
                                 gsems[b])
        for b in range(NBUF):
            j = j0 + b

            @pl.when(j < nj)
            def _(b=b, j=j):
                pltpu.make_async_copy(y_src.at[idxs_v.at[j]], rows_v.at[b],
                                      gsems[b]).wait()
                pltpu.async_copy(rows_v.at[b], acc_sh.at[idxd_v.at[j]],
                                 ssems[b], add=True)
        return carry

    lax.fori_loop(0, (nj + NBUF - 1) // NBUF, outer, 0)
    for b in range(NBUF):
        @pl.when(b < nj)
        def _(b=b):
            pltpu.make_async_copy(rows_v.at[b], acc_sh.at[idxd_v.at[0]],
                                  ssems[b]).wait()


def _copy_out(acc_sh, rows_v, ssems, out_hbm, cid, base, wr):
    nfull = wr // G
    rem = wr - nfull * G
    nchunks = nfull + (1 if rem else 0)

    def chunk(k):
        if k < nfull:
            return base + k * G, G
        return base + nfull * G, rem

    def bufref(b, sz):
        return rows_v.at[b] if sz == G else rows_v.at[b, pl.ds(0, sz)]

    for k in range(nchunks):
        b = k % NBUF
        off, sz = chunk(k)
        if k >= NBUF:
            poff, psz = chunk(k - NBUF)
            pltpu.make_async_copy(bufref(b, psz),
                                  out_hbm.at[cid, pl.ds(poff, psz)],
                                  ssems[b]).wait()
        pltpu.sync_copy(acc_sh.at[pl.ds(off, sz)], bufref(b, sz))
        pltpu.async_copy(bufref(b, sz), out_hbm.at[cid, pl.ds(off, sz)],
                         ssems[b])
    for k in range(max(0, nchunks - NBUF), nchunks):
        b = k % NBUF
        off, sz = chunk(k)
        pltpu.make_async_copy(bufref(b, sz), out_hbm.at[cid, pl.ds(off, sz)],
                              ssems[b]).wait()


def _sc_agg_cols(n, dh, ng):
    nb = ng // 8
    rg = ng - 8 * nb
    win = 8 * _cdiv(nb, NS) + rg
    wr = _row_window(n)

    @functools.partial(
        pl.kernel,
        out_type=jax.ShapeDtypeStruct((NC, n, dh), jnp.float32),
        mesh=_mesh(),
        scratch_types=(
            [pltpu.VMEM((win, G), jnp.int32),
             pltpu.VMEM((win, G), jnp.int32),
             pltpu.VMEM((NBUF, G, dh), jnp.float32)]
            + [pltpu.SemaphoreType.DMA] * (2 * NBUF)
            + [pltpu.VMEM_SHARED((n, dh), jnp.float32)]
        ),
        compiler_params=pltpu.CompilerParams(use_tc_tiling_on_sc=False),
    )
    def agg_kernel(y_hbm, ei_hbm, out_hbm,
                   idxs_v, idxd_v, rows_v, *rest):
        gsems = rest[:NBUF]
        ssems = rest[NBUF:2 * NBUF]
        acc_sh = rest[2 * NBUF]
        cid = lax.axis_index("c")
        sid = lax.axis_index("s")

        _build_zero_rows(rows_v.at[0], dh)
        base = _row_start(sid, n)
        _zero_acc(acc_sh, rows_v.at[0], gsems[0], base, wr)
        plsc.subcore_barrier()

        b0 = (nb * sid) // NS
        b1 = (nb * (sid + 1)) // NS
        c0 = 8 * b0
        nj = 8 * (b1 - b0) + jnp.where(sid == NS - 1, rg, 0)
        pltpu.sync_copy(ei_hbm.at[0, pl.ds(c0, win)], idxs_v)
        pltpu.sync_copy(ei_hbm.at[1, pl.ds(c0, win)], idxd_v)

        @pl.when(cid == 0)
        def _():
            _edge_pipeline(y_hbm.at[0], idxs_v, idxd_v, rows_v,
                           gsems, ssems, acc_sh, nj)

        @pl.when(cid == 1)
        def _():
            _edge_pipeline(y_hbm.at[1], idxs_v, idxd_v, rows_v,
                           gsems, ssems, acc_sh, nj)

        plsc.subcore_barrier()
        _copy_out(acc_sh, rows_v, ssems, out_hbm, cid, base, wr)

    return agg_kernel


def _sc_agg(n, d, ng):
    win = _win_size(ng)
    wr = _row_window(n)

    @functools.partial(
        pl.kernel,
        out_type=jax.ShapeDtypeStruct((NC, n, d), jnp.float32),
        mesh=_mesh(),
        scratch_types=(
            [pltpu.VMEM((win, G), jnp.int32),
             pltpu.VMEM((win, G), jnp.int32),
             pltpu.VMEM((NBUF, G, d), jnp.float32)]
            + [pltpu.SemaphoreType.DMA] * (2 * NBUF)
            + [pltpu.VMEM_SHARED((n, d), jnp.float32)]
        ),
        compiler_params=pltpu.CompilerParams(use_tc_tiling_on_sc=False),
    )
    def agg_kernel(y_hbm, ei_hbm, out_hbm,
                   idxs_v, idxd_v, rows_v, *rest):
        gsems = rest[:NBUF]
        ssems = rest[NBUF:2 * NBUF]
        acc_sh = rest[2 * NBUF]
        cid = lax.axis_index("c")
        sid = lax.axis_index("s")
        wid = sid * NC + cid

        _build_zero_rows(rows_v.at[0], d)
        base = _row_start(sid, n)
        _zero_acc(acc_sh, rows_v.at[0], gsems[0], base, wr)
        plsc.subcore_barrier()

        c0, nj = _group_range(wid, ng)
        pltpu.sync_copy(ei_hbm.at[0, pl.ds(c0, win)], idxs_v)
        pltpu.sync_copy(ei_hbm.at[1, pl.ds(c0, win)], idxd_v)

        _edge_pipeline(y_hbm, idxs_v, idxd_v, rows_v,
                       gsems, ssems, acc_sh, nj)

        plsc.subcore_barrier()
        _copy_out(acc_sh, rows_v, ssems, out_hbm, cid, base, wr)

    return agg_kernel


TCBLK = 1000


def _dinv_from_partials(p_ref):
    deg = p_ref[0, 0] + p_ref[1, 0] + 1.0
    return jnp.transpose(lax.rsqrt(deg), (1, 0))


def _tc_scale(x, degp):
    n, din = x.shape
    dh = din // NC
    blk = TCBLK

    def body(x_ref, p_ref, y_ref):
        dinv = _dinv_from_partials(p_ref)
        y_ref[0] = x_ref[:, :dh] * dinv
        y_ref[1] = x_ref[:, dh:] * dinv

    return pl.pallas_call(
        body,
        grid=(n // blk,),
        in_specs=[
            pl.BlockSpec((blk, din), lambda i: (i, 0)),
            pl.BlockSpec((NC, 1, 1, TCBLK), lambda i: (0, i, 0, 0)),
        ],
        out_specs=pl.BlockSpec((NC, blk, dh), lambda i: (0, i, 0)),
        out_shape=jax.ShapeDtypeStruct((NC, n, dh), jnp.float32),
    )(x, degp)


def _tc_mlp(aggp, y2, degp, W1, b1, W2):
    nc, n, dh = y2.shape
    din = nc * dh
    dhid = W1.shape[1]
    dout = W2.shape[1]
    blk = TCBLK

    def body(a_ref, y_ref, p_ref, w1_ref, b1_ref, w2_ref, z_ref):
        dinv = _dinv_from_partials(p_ref)
        s0 = (a_ref[0] + y_ref[0]) * dinv
        s1 = (a_ref[1] + y_ref[1]) * dinv
        s = jnp.concatenate([s0, s1], axis=1).astype(jnp.bfloat16)
        h = jnp.dot(s, w1_ref[...], preferred_element_type=jnp.float32)
        h = jnp.maximum(h + b1_ref[...], 0.0).astype(jnp.bfloat16)
        z = jnp.dot(h, w2_ref[...], preferred_element_type=jnp.float32)
        z_ref[...] = z * dinv

    return pl.pallas_call(
        body,
        grid=(n // blk,),
        in_specs=[
            pl.BlockSpec((NC, blk, dh), lambda i: (0, i, 0)),
            pl.BlockSpec((NC, blk, dh), lambda i: (0, i, 0)),
            pl.BlockSpec((NC, 1, 1, TCBLK), lambda i: (0, i, 0, 0)),
            pl.BlockSpec((din, dhid), lambda i: (0, 0)),
            pl.BlockSpec((1, dhid), lambda i: (0, 0)),
            pl.BlockSpec((dhid, dout), lambda i: (0, 0)),
        ],
        out_specs=pl.BlockSpec((blk, dout), lambda i: (i, 0)),
        out_shape=jax.ShapeDtypeStruct((n, dout), jnp.float32),
    )(aggp, y2, degp, W1.astype(jnp.bfloat16), b1.reshape(1, dhid),
      W2.astype(jnp.bfloat16))


def _tc_final(cp, z, degp, b2):
    n, dout = z.shape
    blk = TCBLK

    def body(c_ref, z_ref, p_ref, b2_ref, o_ref):
        dinv = _dinv_from_partials(p_ref)
        o = (c_ref[0] + c_ref[1] + z_ref[...]) * dinv + b2_ref[...]
        m = jnp.max(o, axis=1, keepdims=True)
        e = jnp.exp(o - m)
        s = jnp.sum(e, axis=1, keepdims=True)
        o_ref[...] = (o - m) - jnp.log(s)

    return pl.pallas_call(
        body,
        grid=(n // blk,),
        in_specs=[
            pl.BlockSpec((NC, blk, dout), lambda i: (0, i, 0)),
            pl.BlockSpec((blk, dout), lambda i: (i, 0)),
            pl.BlockSpec((NC, 1, 1, TCBLK), lambda i: (0, i, 0, 0)),
            pl.BlockSpec((1, dout), lambda i: (0, 0)),
        ],
        out_specs=pl.BlockSpec((blk, dout), lambda i: (i, 0)),
        out_shape=jax.ShapeDtypeStruct((n, dout), jnp.float32),
    )(cp, z, degp, b2.reshape(1, dout))


def kernel(x, edge_index, W1, b1, W2, b2):
    n, din = x.shape
    e = edge_index.shape[1]
    ng = e // G
    ei3 = edge_index.reshape(2, ng, G)

    degp = _sc_degree(n, ng)(ei3)
    degp4 = degp.reshape(NC, n // TCBLK, 1, TCBLK)
    y2 = _tc_scale(x, degp4)
    aggp = _sc_agg_cols(n, din // NC, ng)(y2, ei3)
    z = _tc_mlp(aggp, y2, degp4, W1, b1, W2)
    cp = _sc_agg(n, W2.shape[1], ng)(z, ei3)
    return _tc_final(cp, z, degp4, b2)


# speedup vs baseline: 49.5949x; 1.0172x over previous
import functools

import jax
import jax.numpy as jnp
from jax import lax
from jax.experimental import pallas as pl
from jax.experimental.pallas import tpu as pltpu
from jax.experimental.pallas import tpu_sc as plsc

NC = 2
NS = 16
NW = NC * NS
LANES = 16
G = 64


def _mesh():
    return plsc.VectorSubcoreMesh(core_axis_name="c", subcore_axis_name="s")


def _cdiv(a, b):
    return (a + b - 1) // b


def _win_size(ng):
    nb = ng // 8
    return 8 * _cdiv(nb, NW) + (ng - 8 * nb)


def _group_range(wid, ng):
    nb = ng // 8
    rg = ng - 8 * nb
    b0 = (nb * wid) // NW
    b1 = (nb * (wid + 1)) // NW
    c0 = 8 * b0
    nj = 8 * (b1 - b0) + jnp.where(wid == NW - 1, rg, 0)
    return c0, nj


def _row_window(n):
    return 8 * _cdiv(n // 8, NS)


def _row_start(sid, n):
    return 8 * (((n // 8) * sid) // NS)


def _sc_degree(n, ng):
    win = _win_size(ng)
    wr = _row_window(n)

    @functools.partial(
        pl.kernel,
        out_type=jax.ShapeDtypeStruct((NC, n), jnp.float32),
        mesh=_mesh(),
        scratch_types=[
            pltpu.VMEM((win, G), jnp.int32),
            pltpu.VMEM((G, LANES), jnp.float32),
            pltpu.VMEM((G, LANES), jnp.float32),
            pltpu.VMEM((G,), jnp.float32),
            pltpu.SemaphoreType.DMA,
            pltpu.VMEM_SHARED((n, LANES), jnp.float32),
        ],
        compiler_params=pltpu.CompilerParams(use_tc_tiling_on_sc=False,
                                             needs_layout_passes=False),
    )
    def deg_kernel(ei_hbm, out_hbm, idx_v, ones_v, buf_v, col_v, sem, acc_sh):
        cid = lax.axis_index("c")
        sid = lax.axis_index("s")
        wid = sid * NC + cid

        def initrow(i, carry):
            ones_v[i, :] = jnp.full((LANES,), 1.0, jnp.float32)
            buf_v[i, :] = jnp.zeros((LANES,), jnp.float32)
            return carry

        lax.fori_loop(0, G, initrow, 0)

        base = _row_start(sid, n)
        nfull = wr // G
        rem = wr - nfull * G
        for k in range(nfull):
            pltpu.sync_copy(buf_v, acc_sh.at[pl.ds(base + k * G, G)])
        if rem:
            pltpu.sync_copy(buf_v.at[pl.ds(0, rem)],
                            acc_sh.at[pl.ds(base + nfull * G, rem)])
        plsc.subcore_barrier()

        c0, nj = _group_range(wid, ng)
        pltpu.sync_copy(ei_hbm.at[1, pl.ds(c0, win)], idx_v)

        def edge_group(j, carry):
            pltpu.async_copy(ones_v, acc_sh.at[idx_v.at[j]], sem, add=True)
            return carry

        lax.fori_loop(0, nj, edge_group, 0)

        def edge_drain(j, carry):
            pltpu.make_async_copy(ones_v, acc_sh.at[idx_v.at[j]], sem).wait()
            return carry

        lax.fori_loop(0, nj, edge_drain, 0)
        plsc.subcore_barrier()

        lane = lax.iota(jnp.int32, LANES)
        zero16 = jnp.zeros((LANES,), jnp.int32)
        nchunks = nfull + (1 if rem else 0)
        for k in range(nchunks):
            off = base + k * G
            sz = G if k < nfull else rem
            pltpu.sync_copy(acc_sh.at[pl.ds(off, sz)],
                            buf_v if sz == G else buf_v.at[pl.ds(0, sz)])
            for j in range(_cdiv(sz, LANES)):
                rowi = jnp.minimum(j * LANES + lane, sz - 1)
                col_v[pl.ds(j * LANES, LANES)] = plsc.load_gather(
                    buf_v, [rowi, zero16])
            pltpu.sync_copy(col_v if sz == G else col_v.at[pl.ds(0, sz)],
                            out_hbm.at[cid, pl.ds(off, sz)])

    return deg_kernel


NBUF = 8


def _build_zero_rows(buf, d):
    def zrow(i, carry):
        for c in range(d // LANES):
            buf[i, pl.ds(c * LANES, LANES)] = jnp.zeros((LANES,), jnp.float32)
        return carry
    lax.fori_loop(0, G, zrow, 0)


def _zero_acc(acc_sh, zbuf, sem, base, wr):
    nfz = wr // G
    rz = wr - nfz * G
    for k in range(nfz):
        pltpu.async_copy(zbuf, acc_sh.at[pl.ds(base + k * G, G)], sem)
    if rz:
        pltpu.async_copy(zbuf.at[pl.ds(0, rz)],
                         acc_sh.at[pl.ds(base + nfz * G, rz)], sem)
    for k in range(nfz):
        pltpu.make_async_copy(zbuf, acc_sh.at[pl.ds(base + k * G, G)],
                              sem).wait()
    if rz:
        pltpu.make_async_copy(zbuf.at[pl.ds(0, rz)],
                              acc_sh.at[pl.ds(base + nfz * G, rz)],
                              sem).wait()


def _edge_pipeline(y_src, idxs_v, idxd_v, rows_v, gsems, ssems, acc_sh, nj,
                   nbuf):
    def outer(g, carry):
        j0 = g * nbuf
        for b in range(nbuf):
            j = j0 + b

            @pl.when(jnp.logical_and(j < nj, j >= nbuf))
            def _(b=b, j=j):
                pltpu.make_async_copy(rows_v.at[b],
                                      acc_sh.at[idxd_v.at[j - nbuf]],
                                      ssems[b]).wait()

            @pl.when(j < nj)
            def _(b=b, j=j):
                pltpu.async_copy(y_src.at[idxs_v.at[j]], rows_v.at[b],
                                 gsems[b])
        for b in range(nbuf):
            j = j0 + b

            @pl.when(j < nj)
            def _(b=b, j=j):
                pltpu.make_async_copy(y_src.at[idxs_v.at[j]], rows_v.at[b],
                                      gsems[b]).wait()
                pltpu.async_copy(rows_v.at[b], acc_sh.at[idxd_v.at[j]],
                                 ssems[b], add=True)
        return carry

    lax.fori_loop(0, (nj + nbuf - 1) // nbuf, outer, 0)
    for b in range(nbuf):
        @pl.when(b < nj)
        def _(b=b):
            pltpu.make_async_copy(rows_v.at[b], acc_sh.at[idxd_v.at[0]],
                                  ssems[b]).wait()


def _copy_out(acc_sh, rows_v, ssems, out_hbm, cid, base, wr, nbuf):
    nfull = wr // G
    rem = wr - nfull * G
    nchunks = nfull + (1 if rem else 0)

    def chunk(k):
        if k < nfull:
            return base + k * G, G
        return base + nfull * G, rem

    def bufref(b, sz):
        return rows_v.at[b] if sz == G else rows_v.at[b, pl.ds(0, sz)]

    for k in range(nchunks):
        b = k % nbuf
        off, sz = chunk(k)
        if k >= nbuf:
            poff, psz = chunk(k - nbuf)
            pltpu.make_async_copy(bufref(b, psz),
                                  out_hbm.at[cid, pl.ds(poff, psz)],
                                  ssems[b]).wait()
        pltpu.sync_copy(acc_sh.at[pl.ds(off, sz)], bufref(b, sz))
        pltpu.async_copy(bufref(b, sz), out_hbm.at[cid, pl.ds(off, sz)],
                         ssems[b])
    for k in range(max(0, nchunks - nbuf), nchunks):
        b = k % nbuf
        off, sz = chunk(k)
        pltpu.make_async_copy(bufref(b, sz), out_hbm.at[cid, pl.ds(off, sz)],
                              ssems[b]).wait()


def _sc_agg(n, d, ng, nbuf=NBUF):
    win = _win_size(ng)
    wr = _row_window(n)

    @functools.partial(
        pl.kernel,
        out_type=jax.ShapeDtypeStruct((NC, n, d), jnp.float32),
        mesh=_mesh(),
        scratch_types=(
            [pltpu.VMEM((win, G), jnp.int32),
             pltpu.VMEM((win, G), jnp.int32),
             pltpu.VMEM((nbuf, G, d), jnp.float32)]
            + [pltpu.SemaphoreType.DMA] * (2 * nbuf)
            + [pltpu.VMEM_SHARED((n, d), jnp.float32)]
        ),
        compiler_params=pltpu.CompilerParams(use_tc_tiling_on_sc=False),
    )
    def agg_kernel(y_hbm, ei_hbm, out_hbm,
                   idxs_v, idxd_v, rows_v, *rest):
        gsems = rest[:nbuf]
        ssems = rest[nbuf:2 * nbuf]
        acc_sh = rest[2 * nbuf]
        cid = lax.axis_index("c")
        sid = lax.axis_index("s")
        wid = sid * NC + cid

        _build_zero_rows(rows_v.at[0], d)
        base = _row_start(sid, n)
        _zero_acc(acc_sh, rows_v.at[0], gsems[0], base, wr)
        plsc.subcore_barrier()

        c0, nj = _group_range(wid, ng)
        pltpu.sync_copy(ei_hbm.at[0, pl.ds(c0, win)], idxs_v)
        pltpu.sync_copy(ei_hbm.at[1, pl.ds(c0, win)], idxd_v)

        _edge_pipeline(y_hbm, idxs_v, idxd_v, rows_v,
                       gsems, ssems, acc_sh, nj, nbuf)

        plsc.subcore_barrier()
        _copy_out(acc_sh, rows_v, ssems, out_hbm, cid, base, wr, nbuf)

    return agg_kernel


TCBLK = 1000


def _dinv_from_partials(p_ref):
    deg = p_ref[0, 0] + p_ref[1, 0] + 1.0
    return jnp.transpose(lax.rsqrt(deg), (1, 0))


def _tc_scale(x, degp):
    n, din = x.shape
    blk = TCBLK

    def body(x_ref, p_ref, y_ref):
        dinv = _dinv_from_partials(p_ref)
        y_ref[...] = x_ref[...] * dinv

    return pl.pallas_call(
        body,
        grid=(n // blk,),
        in_specs=[
            pl.BlockSpec((blk, din), lambda i: (i, 0)),
            pl.BlockSpec((NC, 1, 1, TCBLK), lambda i: (0, i, 0, 0)),
        ],
        out_specs=pl.BlockSpec((blk, din), lambda i: (i, 0)),
        out_shape=jax.ShapeDtypeStruct((n, din), jnp.float32),
    )(x, degp)


def _tc_mlp(aggp, y, degp, W1, b1, W2):
    n, din = y.shape
    dhid = W1.shape[1]
    dout = W2.shape[1]
    blk = TCBLK

    def body(a_ref, y_ref, p_ref, w1_ref, b1_ref, w2_ref, z_ref):
        dinv = _dinv_from_partials(p_ref)
        s = (a_ref[0] + a_ref[1] + y_ref[...]) * dinv
        s = s.astype(jnp.bfloat16)
        h = jnp.dot(s, w1_ref[...], preferred_element_type=jnp.float32)
        h = jnp.maximum(h + b1_ref[...], 0.0).astype(jnp.bfloat16)
        z = jnp.dot(h, w2_ref[...], preferred_element_type=jnp.float32)
        z_ref[...] = z * dinv

    return pl.pallas_call(
        body,
        grid=(n // blk,),
        in_specs=[
            pl.BlockSpec((NC, blk, din), lambda i: (0, i, 0)),
            pl.BlockSpec((blk, din), lambda i: (i, 0)),
            pl.BlockSpec((NC, 1, 1, TCBLK), lambda i: (0, i, 0, 0)),
            pl.BlockSpec((din, dhid), lambda i: (0, 0)),
            pl.BlockSpec((1, dhid), lambda i: (0, 0)),
            pl.BlockSpec((dhid, dout), lambda i: (0, 0)),
        ],
        out_specs=pl.BlockSpec((blk, dout), lambda i: (i, 0)),
        out_shape=jax.ShapeDtypeStruct((n, dout), jnp.float32),
    )(aggp, y, degp, W1.astype(jnp.bfloat16), b1.reshape(1, dhid),
      W2.astype(jnp.bfloat16))


def _tc_final(cp, z, degp, b2):
    n, dout = z.shape
    blk = TCBLK

    def body(c_ref, z_ref, p_ref, b2_ref, o_ref):
        dinv = _dinv_from_partials(p_ref)
        o = (c_ref[0] + c_ref[1] + z_ref[...]) * dinv + b2_ref[...]
        m = jnp.max(o, axis=1, keepdims=True)
        e = jnp.exp(o - m)
        s = jnp.sum(e, axis=1, keepdims=True)
        o_ref[...] = (o - m) - jnp.log(s)

    return pl.pallas_call(
        body,
        grid=(n // blk,),
        in_specs=[
            pl.BlockSpec((NC, blk, dout), lambda i: (0, i, 0)),
            pl.BlockSpec((blk, dout), lambda i: (i, 0)),
            pl.BlockSpec((NC, 1, 1, TCBLK), lambda i: (0, i, 0, 0)),
            pl.BlockSpec((1, dout), lambda i: (0, 0)),
        ],
        out_specs=pl.BlockSpec((blk, dout), lambda i: (i, 0)),
        out_shape=jax.ShapeDtypeStruct((n, dout), jnp.float32),
    )(cp, z, degp, b2.reshape(1, dout))


def kernel(x, edge_index, W1, b1, W2, b2):
    n, din = x.shape
    e = edge_index.shape[1]
    ng = e // G
    ei3 = edge_index.reshape(2, ng, G)

    degp = _sc_degree(n, ng)(ei3)
    degp4 = degp.reshape(NC, n // TCBLK, 1, TCBLK)
    y = _tc_scale(x, degp4)
    aggp = _sc_agg(n, din, ng, nbuf=3)(y, ei3)
    z = _tc_mlp(aggp, y, degp4, W1, b1, W2)
    cp = _sc_agg(n, W2.shape[1], ng)(z, ei3)
    return _tc_final(cp, z, degp4, b2)


# speedup vs baseline: 51.3455x; 1.0353x over previous
import functools

import jax
import jax.numpy as jnp
from jax import lax
from jax.experimental import pallas as pl
from jax.experimental.pallas import tpu as pltpu
from jax.experimental.pallas import tpu_sc as plsc

NC = 2
NS = 16
NW = NC * NS
LANES = 16
G = 64


def _mesh():
    return plsc.VectorSubcoreMesh(core_axis_name="c", subcore_axis_name="s")


def _cdiv(a, b):
    return (a + b - 1) // b


def _win_size(ng):
    nb = ng // 8
    return 8 * _cdiv(nb, NW) + (ng - 8 * nb)


def _group_range(wid, ng):
    nb = ng // 8
    rg = ng - 8 * nb
    b0 = (nb * wid) // NW
    b1 = (nb * (wid + 1)) // NW
    c0 = 8 * b0
    nj = 8 * (b1 - b0) + jnp.where(wid == NW - 1, rg, 0)
    return c0, nj


def _row_window(n):
    return 8 * _cdiv(n // 8, NS)


def _row_start(sid, n):
    return 8 * (((n // 8) * sid) // NS)


def _sc_degree(n, ng):
    win = _win_size(ng)
    wr = _row_window(n)

    @functools.partial(
        pl.kernel,
        out_type=jax.ShapeDtypeStruct((NC, n), jnp.float32),
        mesh=_mesh(),
        scratch_types=[
            pltpu.VMEM((win, G), jnp.int32),
            pltpu.VMEM((G, LANES), jnp.float32),
            pltpu.VMEM((G, LANES), jnp.float32),
            pltpu.VMEM((G,), jnp.float32),
            pltpu.SemaphoreType.DMA,
            pltpu.VMEM_SHARED((n, LANES), jnp.float32),
        ],
        compiler_params=pltpu.CompilerParams(use_tc_tiling_on_sc=False,
                                             needs_layout_passes=False),
    )
    def deg_kernel(ei_hbm, out_hbm, idx_v, ones_v, buf_v, col_v, sem, acc_sh):
        cid = lax.axis_index("c")
        sid = lax.axis_index("s")
        wid = sid * NC + cid

        def initrow(i, carry):
            ones_v[i, :] = jnp.full((LANES,), 1.0, jnp.float32)
            buf_v[i, :] = jnp.zeros((LANES,), jnp.float32)
            return carry

        lax.fori_loop(0, G, initrow, 0)

        base = _row_start(sid, n)
        nfull = wr // G
        rem = wr - nfull * G
        for k in range(nfull):
            pltpu.sync_copy(buf_v, acc_sh.at[pl.ds(base + k * G, G)])
        if rem:
            pltpu.sync_copy(buf_v.at[pl.ds(0, rem)],
                            acc_sh.at[pl.ds(base + nfull * G, rem)])
        plsc.subcore_barrier()

        c0, nj = _group_range(wid, ng)
        pltpu.sync_copy(ei_hbm.at[1, pl.ds(c0, win)], idx_v)

        def edge_group(j, carry):
            pltpu.async_copy(ones_v, acc_sh.at[idx_v.at[j]], sem, add=True)
            return carry

        lax.fori_loop(0, nj, edge_group, 0)

        def edge_drain(j, carry):
            pltpu.make_async_copy(ones_v, acc_sh.at[idx_v.at[j]], sem).wait()
            return carry

        lax.fori_loop(0, nj, edge_drain, 0)
        plsc.subcore_barrier()

        lane = lax.iota(jnp.int32, LANES)
        zero16 = jnp.zeros((LANES,), jnp.int32)
        nchunks = nfull + (1 if rem else 0)
        for k in range(nchunks):
            off = base + k * G
            sz = G if k < nfull else rem
            pltpu.sync_copy(acc_sh.at[pl.ds(off, sz)],
                            buf_v if sz == G else buf_v.at[pl.ds(0, sz)])
            for j in range(_cdiv(sz, LANES)):
                rowi = jnp.minimum(j * LANES + lane, sz - 1)
                col_v[pl.ds(j * LANES, LANES)] = plsc.load_gather(
                    buf_v, [rowi, zero16])
            pltpu.sync_copy(col_v if sz == G else col_v.at[pl.ds(0, sz)],
                            out_hbm.at[cid, pl.ds(off, sz)])

    return deg_kernel


NBUF = 8


def _zero_acc(acc_sh, zbuf, sem, base, wr):
    nfz = wr // G
    rz = wr - nfz * G
    for k in range(nfz):
        pltpu.async_copy(zbuf, acc_sh.at[pl.ds(base + k * G, G)], sem)
    if rz:
        pltpu.async_copy(zbuf.at[pl.ds(0, rz)],
                         acc_sh.at[pl.ds(base + nfz * G, rz)], sem)
    for k in range(nfz):
        pltpu.make_async_copy(zbuf, acc_sh.at[pl.ds(base + k * G, G)],
                              sem).wait()
    if rz:
        pltpu.make_async_copy(zbuf.at[pl.ds(0, rz)],
                              acc_sh.at[pl.ds(base + nfz * G, rz)],
                              sem).wait()


def _edge_pipeline(y_src, idxs_v, idxd_v, rows_v, gsems, ssems, acc_sh, nj,
                   nbuf):
    def outer(g, carry):
        j0 = g * nbuf
        for b in range(nbuf):
            j = j0 + b

            @pl.when(jnp.logical_and(j < nj, j >= nbuf))
            def _(b=b, j=j):
                pltpu.make_async_copy(rows_v.at[b],
                                      acc_sh.at[idxd_v.at[j - nbuf]],
                                      ssems[b]).wait()

            @pl.when(j < nj)
            def _(b=b, j=j):
                pltpu.async_copy(y_src.at[idxs_v.at[j]], rows_v.at[b],
                                 gsems[b])
        for b in range(nbuf):
            j = j0 + b

            @pl.when(j < nj)
            def _(b=b, j=j):
                pltpu.make_async_copy(y_src.at[idxs_v.at[j]], rows_v.at[b],
                                      gsems[b]).wait()
                pltpu.async_copy(rows_v.at[b], acc_sh.at[idxd_v.at[j]],
                                 ssems[b], add=True)
        return carry

    lax.fori_loop(0, (nj + nbuf - 1) // nbuf, outer, 0)
    for b in range(nbuf):
        @pl.when(b < nj)
        def _(b=b):
            pltpu.make_async_copy(rows_v.at[b], acc_sh.at[idxd_v.at[0]],
                                  ssems[b]).wait()


def _copy_out(acc_sh, rows_v, ssems, out_hbm, cid, base, wr, nbuf):
    nfull = wr // G
    rem = wr - nfull * G
    nchunks = nfull + (1 if rem else 0)

    def chunk(k):
        if k < nfull:
            return base + k * G, G
        return base + nfull * G, rem

    def bufref(b, sz):
        return rows_v.at[b] if sz == G else rows_v.at[b, pl.ds(0, sz)]

    for k in range(nchunks):
        b = k % nbuf
        off, sz = chunk(k)
        if k >= nbuf:
            poff, psz = chunk(k - nbuf)
            pltpu.make_async_copy(bufref(b, psz),
                                  out_hbm.at[cid, pl.ds(poff, psz)],
                                  ssems[b]).wait()
        pltpu.sync_copy(acc_sh.at[pl.ds(off, sz)], bufref(b, sz))
        pltpu.async_copy(bufref(b, sz), out_hbm.at[cid, pl.ds(off, sz)],
                         ssems[b])
    for k in range(max(0, nchunks - nbuf), nchunks):
        b = k % nbuf
        off, sz = chunk(k)
        pltpu.make_async_copy(bufref(b, sz), out_hbm.at[cid, pl.ds(off, sz)],
                              ssems[b]).wait()


def _sc_agg(n, d, ng, nbuf=NBUF, dtype=jnp.float32):
    win = _win_size(ng)
    wr = _row_window(n)

    @functools.partial(
        pl.kernel,
        out_type=jax.ShapeDtypeStruct((NC, n, d), dtype),
        mesh=_mesh(),
        scratch_types=(
            [pltpu.VMEM((win, G), jnp.int32),
             pltpu.VMEM((win, G), jnp.int32),
             pltpu.VMEM((nbuf, G, d), dtype)]
            + [pltpu.SemaphoreType.DMA] * (2 * nbuf)
            + [pltpu.VMEM_SHARED((n, d), dtype)]
        ),
        compiler_params=pltpu.CompilerParams(use_tc_tiling_on_sc=False),
    )
    def agg_kernel(y_hbm, ei_hbm, z0_hbm, out_hbm,
                   idxs_v, idxd_v, rows_v, *rest):
        gsems = rest[:nbuf]
        ssems = rest[nbuf:2 * nbuf]
        acc_sh = rest[2 * nbuf]
        cid = lax.axis_index("c")
        sid = lax.axis_index("s")
        wid = sid * NC + cid

        pltpu.sync_copy(z0_hbm, rows_v.at[0])
        base = _row_start(sid, n)
        _zero_acc(acc_sh, rows_v.at[0], gsems[0], base, wr)
        plsc.subcore_barrier()

        c0, nj = _group_range(wid, ng)
        pltpu.sync_copy(ei_hbm.at[0, pl.ds(c0, win)], idxs_v)
        pltpu.sync_copy(ei_hbm.at[1, pl.ds(c0, win)], idxd_v)

        _edge_pipeline(y_hbm, idxs_v, idxd_v, rows_v,
                       gsems, ssems, acc_sh, nj, nbuf)

        plsc.subcore_barrier()
        _copy_out(acc_sh, rows_v, ssems, out_hbm, cid, base, wr, nbuf)

    return agg_kernel


TCBLK = 1000


def _dinv_from_partials(p_ref):
    deg = p_ref[0, 0] + p_ref[1, 0] + 1.0
    return jnp.transpose(lax.rsqrt(deg), (1, 0))


def _tc_scale(x, degp):
    n, din = x.shape
    blk = TCBLK

    def body(x_ref, p_ref, y_ref):
        dinv = _dinv_from_partials(p_ref)
        y_ref[...] = (x_ref[...] * dinv).astype(jnp.bfloat16)

    return pl.pallas_call(
        body,
        grid=(n // blk,),
        in_specs=[
            pl.BlockSpec((blk, din), lambda i: (i, 0)),
            pl.BlockSpec((NC, 1, 1, TCBLK), lambda i: (0, i, 0, 0)),
        ],
        out_specs=pl.BlockSpec((blk, din), lambda i: (i, 0)),
        out_shape=jax.ShapeDtypeStruct((n, din), jnp.bfloat16),
    )(x, degp)


def _tc_mlp(aggp, y, degp, W1, b1, W2):
    n, din = y.shape
    dhid = W1.shape[1]
    dout = W2.shape[1]
    blk = TCBLK

    def body(a_ref, y_ref, p_ref, w1_ref, b1_ref, w2_ref, z_ref):
        dinv = _dinv_from_partials(p_ref)
        s = (a_ref[0].astype(jnp.float32) + a_ref[1].astype(jnp.float32)
             + y_ref[...].astype(jnp.float32)) * dinv
        s = s.astype(jnp.bfloat16)
        h = jnp.dot(s, w1_ref[...], preferred_element_type=jnp.float32)
        h = jnp.maximum(h + b1_ref[...], 0.0).astype(jnp.bfloat16)
        z = jnp.dot(h, w2_ref[...], preferred_element_type=jnp.float32)
        z_ref[...] = z * dinv

    return pl.pallas_call(
        body,
        grid=(n // blk,),
        in_specs=[
            pl.BlockSpec((NC, blk, din), lambda i: (0, i, 0)),
            pl.BlockSpec((blk, din), lambda i: (i, 0)),
            pl.BlockSpec((NC, 1, 1, TCBLK), lambda i: (0, i, 0, 0)),
            pl.BlockSpec((din, dhid), lambda i: (0, 0)),
            pl.BlockSpec((1, dhid), lambda i: (0, 0)),
            pl.BlockSpec((dhid, dout), lambda i: (0, 0)),
        ],
        out_specs=pl.BlockSpec((blk, dout), lambda i: (i, 0)),
        out_shape=jax.ShapeDtypeStruct((n, dout), jnp.float32),
    )(aggp, y, degp, W1.astype(jnp.bfloat16), b1.reshape(1, dhid),
      W2.astype(jnp.bfloat16))


def _tc_final(cp, z, degp, b2):
    n, dout = z.shape
    blk = TCBLK

    def body(c_ref, z_ref, p_ref, b2_ref, o_ref):
        dinv = _dinv_from_partials(p_ref)
        o = (c_ref[0] + c_ref[1] + z_ref[...]) * dinv + b2_ref[...]
        m = jnp.max(o, axis=1, keepdims=True)
        e = jnp.exp(o - m)
        s = jnp.sum(e, axis=1, keepdims=True)
        o_ref[...] = (o - m) - jnp.log(s)

    return pl.pallas_call(
        body,
        grid=(n // blk,),
        in_specs=[
            pl.BlockSpec((NC, blk, dout), lambda i: (0, i, 0)),
            pl.BlockSpec((blk, dout), lambda i: (i, 0)),
            pl.BlockSpec((NC, 1, 1, TCBLK), lambda i: (0, i, 0, 0)),
            pl.BlockSpec((1, dout), lambda i: (0, 0)),
        ],
        out_specs=pl.BlockSpec((blk, dout), lambda i: (i, 0)),
        out_shape=jax.ShapeDtypeStruct((n, dout), jnp.float32),
    )(cp, z, degp, b2.reshape(1, dout))


def kernel(x, edge_index, W1, b1, W2, b2):
    n, din = x.shape
    e = edge_index.shape[1]
    ng = e // G
    ei3 = edge_index.reshape(2, ng, G)

    degp = _sc_degree(n, ng)(ei3)
    degp4 = degp.reshape(NC, n // TCBLK, 1, TCBLK)
    y = _tc_scale(x, degp4)
    aggp = _sc_agg(n, din, ng, nbuf=3, dtype=jnp.bfloat16)(
        y, ei3, jnp.zeros((G, din), jnp.bfloat16))
    z = _tc_mlp(aggp, y, degp4, W1, b1, W2)
    cp = _sc_agg(n, W2.shape[1], ng)(z, ei3, jnp.zeros((G, 16), jnp.float32))
    return _tc_final(cp, z, degp4, b2)


# speedup vs baseline: 57.4398x; 1.1187x over previous
import functools

import jax
import jax.numpy as jnp
from jax import lax
from jax.experimental import pallas as pl
from jax.experimental.pallas import tpu as pltpu
from jax.experimental.pallas import tpu_sc as plsc

NC = 2
NS = 16
NW = NC * NS
LANES = 16
G = 128


def _mesh():
    return plsc.VectorSubcoreMesh(core_axis_name="c", subcore_axis_name="s")


def _cdiv(a, b):
    return (a + b - 1) // b


def _win_size(ng):
    nb = ng // 8
    return 8 * _cdiv(nb, NW) + (ng - 8 * nb)


def _group_range(wid, ng):
    nb = ng // 8
    rg = ng - 8 * nb
    b0 = (nb * wid) // NW
    b1 = (nb * (wid + 1)) // NW
    c0 = 8 * b0
    nj = 8 * (b1 - b0) + jnp.where(wid == NW - 1, rg, 0)
    return c0, nj


def _row_window(n):
    return 8 * _cdiv(n // 8, NS)


def _row_start(sid, n):
    return 8 * (((n // 8) * sid) // NS)


def _sc_degree(n, ng):
    win = _win_size(ng)
    wr = _row_window(n)

    @functools.partial(
        pl.kernel,
        out_type=jax.ShapeDtypeStruct((NC, n), jnp.float32),
        mesh=_mesh(),
        scratch_types=[
            pltpu.VMEM((win, G), jnp.int32),
            pltpu.VMEM((G, LANES), jnp.float32),
            pltpu.VMEM((G, LANES), jnp.float32),
            pltpu.VMEM((G,), jnp.float32),
            pltpu.SemaphoreType.DMA,
            pltpu.VMEM_SHARED((n, LANES), jnp.float32),
        ],
        compiler_params=pltpu.CompilerParams(use_tc_tiling_on_sc=False,
                                             needs_layout_passes=False),
    )
    def deg_kernel(ei_hbm, out_hbm, idx_v, ones_v, buf_v, col_v, sem, acc_sh):
        cid = lax.axis_index("c")
        sid = lax.axis_index("s")
        wid = sid * NC + cid

        def initrow(i, carry):
            ones_v[i, :] = jnp.full((LANES,), 1.0, jnp.float32)
            buf_v[i, :] = jnp.zeros((LANES,), jnp.float32)
            return carry

        lax.fori_loop(0, G, initrow, 0)

        base = _row_start(sid, n)
        nfull = wr // G
        rem = wr - nfull * G
        for k in range(nfull):
            pltpu.sync_copy(buf_v, acc_sh.at[pl.ds(base + k * G, G)])
        if rem:
            pltpu.sync_copy(buf_v.at[pl.ds(0, rem)],
                            acc_sh.at[pl.ds(base + nfull * G, rem)])
        plsc.subcore_barrier()

        c0, nj = _group_range(wid, ng)
        pltpu.sync_copy(ei_hbm.at[1, pl.ds(c0, win)], idx_v)

        def edge_group(j, carry):
            pltpu.async_copy(ones_v, acc_sh.at[idx_v.at[j]], sem, add=True)
            return carry

        lax.fori_loop(0, nj, edge_group, 0)

        def edge_drain(j, carry):
            pltpu.make_async_copy(ones_v, acc_sh.at[idx_v.at[j]], sem).wait()
            return carry

        lax.fori_loop(0, nj, edge_drain, 0)
        plsc.subcore_barrier()

        lane = lax.iota(jnp.int32, LANES)
        zero16 = jnp.zeros((LANES,), jnp.int32)
        nchunks = nfull + (1 if rem else 0)
        for k in range(nchunks):
            off = base + k * G
            sz = G if k < nfull else rem
            pltpu.sync_copy(acc_sh.at[pl.ds(off, sz)],
                            buf_v if sz == G else buf_v.at[pl.ds(0, sz)])
            for j in range(_cdiv(sz, LANES)):
                rowi = jnp.minimum(j * LANES + lane, sz - 1)
                col_v[pl.ds(j * LANES, LANES)] = plsc.load_gather(
                    buf_v, [rowi, zero16])
            pltpu.sync_copy(col_v if sz == G else col_v.at[pl.ds(0, sz)],
                            out_hbm.at[cid, pl.ds(off, sz)])

    return deg_kernel


NBUF = 8


def _zero_acc(acc_sh, zbuf, sem, base, wr):
    nfz = wr // G
    rz = wr - nfz * G
    for k in range(nfz):
        pltpu.async_copy(zbuf, acc_sh.at[pl.ds(base + k * G, G)], sem)
    if rz:
        pltpu.async_copy(zbuf.at[pl.ds(0, rz)],
                         acc_sh.at[pl.ds(base + nfz * G, rz)], sem)
    for k in range(nfz):
        pltpu.make_async_copy(zbuf, acc_sh.at[pl.ds(base + k * G, G)],
                              sem).wait()
    if rz:
        pltpu.make_async_copy(zbuf.at[pl.ds(0, rz)],
                              acc_sh.at[pl.ds(base + nfz * G, rz)],
                              sem).wait()


def _edge_pipeline(y_src, idxs_v, idxd_v, rows_v, gsems, ssems, acc_sh, nj,
                   nbuf):
    def outer(g, carry):
        j0 = g * nbuf
        for b in range(nbuf):
            j = j0 + b

            @pl.when(jnp.logical_and(j < nj, j >= nbuf))
            def _(b=b, j=j):
                pltpu.make_async_copy(rows_v.at[b],
                                      acc_sh.at[idxd_v.at[j - nbuf]],
                                      ssems[b]).wait()

            @pl.when(j < nj)
            def _(b=b, j=j):
                pltpu.async_copy(y_src.at[idxs_v.at[j]], rows_v.at[b],
                                 gsems[b])
        for b in range(nbuf):
            j = j0 + b

            @pl.when(j < nj)
            def _(b=b, j=j):
                pltpu.make_async_copy(y_src.at[idxs_v.at[j]], rows_v.at[b],
                                      gsems[b]).wait()
                pltpu.async_copy(rows_v.at[b], acc_sh.at[idxd_v.at[j]],
                                 ssems[b], add=True)
        return carry

    lax.fori_loop(0, (nj + nbuf - 1) // nbuf, outer, 0)
    for b in range(nbuf):
        @pl.when(b < nj)
        def _(b=b):
            pltpu.make_async_copy(rows_v.at[b], acc_sh.at[idxd_v.at[0]],
                                  ssems[b]).wait()


def _copy_out(acc_sh, rows_v, ssems, out_hbm, cid, base, wr, nbuf):
    nfull = wr // G
    rem = wr - nfull * G
    nchunks = nfull + (1 if rem else 0)

    def chunk(k):
        if k < nfull:
            return base + k * G, G
        return base + nfull * G, rem

    def bufref(b, sz):
        return rows_v.at[b] if sz == G else rows_v.at[b, pl.ds(0, sz)]

    for k in range(nchunks):
        b = k % nbuf
        off, sz = chunk(k)
        if k >= nbuf:
            poff, psz = chunk(k - nbuf)
            pltpu.make_async_copy(bufref(b, psz),
                                  out_hbm.at[cid, pl.ds(poff, psz)],
                                  ssems[b]).wait()
        pltpu.sync_copy(acc_sh.at[pl.ds(off, sz)], bufref(b, sz))
        pltpu.async_copy(bufref(b, sz), out_hbm.at[cid, pl.ds(off, sz)],
                         ssems[b])
    for k in range(max(0, nchunks - nbuf), nchunks):
        b = k % nbuf
        off, sz = chunk(k)
        pltpu.make_async_copy(bufref(b, sz), out_hbm.at[cid, pl.ds(off, sz)],
                              ssems[b]).wait()


def _sc_agg(n, d, ng, nbuf=NBUF, dtype=jnp.float32):
    win = _win_size(ng)
    wr = _row_window(n)

    @functools.partial(
        pl.kernel,
        out_type=jax.ShapeDtypeStruct((NC, n, d), dtype),
        mesh=_mesh(),
        scratch_types=(
            [pltpu.VMEM((win, G), jnp.int32),
             pltpu.VMEM((win, G), jnp.int32),
             pltpu.VMEM((nbuf, G, d), dtype)]
            + [pltpu.SemaphoreType.DMA] * (2 * nbuf)
            + [pltpu.VMEM_SHARED((n, d), dtype)]
        ),
        compiler_params=pltpu.CompilerParams(use_tc_tiling_on_sc=False),
    )
    def agg_kernel(y_hbm, ei_hbm, z0_hbm, out_hbm,
                   idxs_v, idxd_v, rows_v, *rest):
        gsems = rest[:nbuf]
        ssems = rest[nbuf:2 * nbuf]
        acc_sh = rest[2 * nbuf]
        cid = lax.axis_index("c")
        sid = lax.axis_index("s")
        wid = sid * NC + cid

        pltpu.sync_copy(z0_hbm, rows_v.at[0])
        base = _row_start(sid, n)
        _zero_acc(acc_sh, rows_v.at[0], gsems[0], base, wr)
        plsc.subcore_barrier()

        c0, nj = _group_range(wid, ng)
        pltpu.sync_copy(ei_hbm.at[0, pl.ds(c0, win)], idxs_v)
        pltpu.sync_copy(ei_hbm.at[1, pl.ds(c0, win)], idxd_v)

        _edge_pipeline(y_hbm, idxs_v, idxd_v, rows_v,
                       gsems, ssems, acc_sh, nj, nbuf)

        plsc.subcore_barrier()
        _copy_out(acc_sh, rows_v, ssems, out_hbm, cid, base, wr, nbuf)

    return agg_kernel


TCBLK = 1000


def _dinv_from_partials(p_ref):
    deg = p_ref[0, 0] + p_ref[1, 0] + 1.0
    return jnp.transpose(lax.rsqrt(deg), (1, 0))


def _tc_scale(x, degp):
    n, din = x.shape
    blk = TCBLK

    def body(x_ref, p_ref, y_ref):
        dinv = _dinv_from_partials(p_ref)
        y_ref[...] = (x_ref[...] * dinv).astype(jnp.bfloat16)

    return pl.pallas_call(
        body,
        grid=(n // blk,),
        in_specs=[
            pl.BlockSpec((blk, din), lambda i: (i, 0)),
            pl.BlockSpec((NC, 1, 1, TCBLK), lambda i: (0, i, 0, 0)),
        ],
        out_specs=pl.BlockSpec((blk, din), lambda i: (i, 0)),
        out_shape=jax.ShapeDtypeStruct((n, din), jnp.bfloat16),
    )(x, degp)


def _tc_mlp(aggp, y, degp, W1, b1, W2):
    n, din = y.shape
    dhid = W1.shape[1]
    dout = W2.shape[1]
    blk = TCBLK

    def body(a_ref, y_ref, p_ref, w1_ref, b1_ref, w2_ref, z_ref):
        dinv = _dinv_from_partials(p_ref)
        s = (a_ref[0].astype(jnp.float32) + a_ref[1].astype(jnp.float32)
             + y_ref[...].astype(jnp.float32)) * dinv
        s = s.astype(jnp.bfloat16)
        h = jnp.dot(s, w1_ref[...], preferred_element_type=jnp.float32)
        h = jnp.maximum(h + b1_ref[...], 0.0).astype(jnp.bfloat16)
        z = jnp.dot(h, w2_ref[...], preferred_element_type=jnp.float32)
        z_ref[...] = z * dinv

    return pl.pallas_call(
        body,
        grid=(n // blk,),
        in_specs=[
            pl.BlockSpec((NC, blk, din), lambda i: (0, i, 0)),
            pl.BlockSpec((blk, din), lambda i: (i, 0)),
            pl.BlockSpec((NC, 1, 1, TCBLK), lambda i: (0, i, 0, 0)),
            pl.BlockSpec((din, dhid), lambda i: (0, 0)),
            pl.BlockSpec((1, dhid), lambda i: (0, 0)),
            pl.BlockSpec((dhid, dout), lambda i: (0, 0)),
        ],
        out_specs=pl.BlockSpec((blk, dout), lambda i: (i, 0)),
        out_shape=jax.ShapeDtypeStruct((n, dout), jnp.float32),
    )(aggp, y, degp, W1.astype(jnp.bfloat16), b1.reshape(1, dhid),
      W2.astype(jnp.bfloat16))


def _tc_final(cp, z, degp, b2):
    n, dout = z.shape
    blk = TCBLK

    def body(c_ref, z_ref, p_ref, b2_ref, o_ref):
        dinv = _dinv_from_partials(p_ref)
        o = (c_ref[0] + c_ref[1] + z_ref[...]) * dinv + b2_ref[...]
        m = jnp.max(o, axis=1, keepdims=True)
        e = jnp.exp(o - m)
        s = jnp.sum(e, axis=1, keepdims=True)
        o_ref[...] = (o - m) - jnp.log(s)

    return pl.pallas_call(
        body,
        grid=(n // blk,),
        in_specs=[
            pl.BlockSpec((NC, blk, dout), lambda i: (0, i, 0)),
            pl.BlockSpec((blk, dout), lambda i: (i, 0)),
            pl.BlockSpec((NC, 1, 1, TCBLK), lambda i: (0, i, 0, 0)),
            pl.BlockSpec((1, dout), lambda i: (0, 0)),
        ],
        out_specs=pl.BlockSpec((blk, dout), lambda i: (i, 0)),
        out_shape=jax.ShapeDtypeStruct((n, dout), jnp.float32),
    )(cp, z, degp, b2.reshape(1, dout))


def kernel(x, edge_index, W1, b1, W2, b2):
    n, din = x.shape
    e = edge_index.shape[1]
    ng = e // G
    ei3 = edge_index.reshape(2, ng, G)

    degp = _sc_degree(n, ng)(ei3)
    degp4 = degp.reshape(NC, n // TCBLK, 1, TCBLK)
    y = _tc_scale(x, degp4)
    aggp = _sc_agg(n, din, ng, nbuf=6, dtype=jnp.bfloat16)(
        y, ei3, jnp.zeros((G, din), jnp.bfloat16))
    z = _tc_mlp(aggp, y, degp4, W1, b1, W2)
    cp = _sc_agg(n, W2.shape[1], ng)(z, ei3, jnp.zeros((G, 16), jnp.float32))
    return _tc_final(cp, z, degp4, b2)


# speedup vs baseline: 57.9076x; 1.0081x over previous
import functools

import jax
import jax.numpy as jnp
from jax import lax
from jax.experimental import pallas as pl
from jax.experimental.pallas import tpu as pltpu
from jax.experimental.pallas import tpu_sc as plsc

NC = 2
NS = 16
NW = NC * NS
LANES = 16
G = 128


def _mesh():
    return plsc.VectorSubcoreMesh(core_axis_name="c", subcore_axis_name="s")


def _cdiv(a, b):
    return (a + b - 1) // b


def _win_size(ng):
    nb = ng // 8
    return 8 * _cdiv(nb, NW) + (ng - 8 * nb)


def _group_range(wid, ng):
    nb = ng // 8
    rg = ng - 8 * nb
    b0 = (nb * wid) // NW
    b1 = (nb * (wid + 1)) // NW
    c0 = 8 * b0
    nj = 8 * (b1 - b0) + jnp.where(wid == NW - 1, rg, 0)
    return c0, nj


def _row_window(n):
    return 8 * _cdiv(n // 8, NS)


def _row_start(sid, n):
    return 8 * (((n // 8) * sid) // NS)


def _sc_degree(n, ng):
    win = _win_size(ng)
    wr = _row_window(n)

    @functools.partial(
        pl.kernel,
        out_type=jax.ShapeDtypeStruct((NC, n), jnp.float32),
        mesh=_mesh(),
        scratch_types=[
            pltpu.VMEM((win, G), jnp.int32),
            pltpu.VMEM((G, LANES), jnp.float32),
            pltpu.VMEM((G, LANES), jnp.float32),
            pltpu.VMEM((G,), jnp.float32),
            pltpu.SemaphoreType.DMA,
            pltpu.VMEM_SHARED((n, LANES), jnp.float32),
        ],
        compiler_params=pltpu.CompilerParams(use_tc_tiling_on_sc=False,
                                             needs_layout_passes=False),
    )
    def deg_kernel(ei_hbm, out_hbm, idx_v, ones_v, buf_v, col_v, sem, acc_sh):
        cid = lax.axis_index("c")
        sid = lax.axis_index("s")
        wid = sid * NC + cid

        def initrow(i, carry):
            ones_v[i, :] = jnp.full((LANES,), 1.0, jnp.float32)
            buf_v[i, :] = jnp.zeros((LANES,), jnp.float32)
            return carry

        lax.fori_loop(0, G, initrow, 0)

        base = _row_start(sid, n)
        nfull = wr // G
        rem = wr - nfull * G
        for k in range(nfull):
            pltpu.sync_copy(buf_v, acc_sh.at[pl.ds(base + k * G, G)])
        if rem:
            pltpu.sync_copy(buf_v.at[pl.ds(0, rem)],
                            acc_sh.at[pl.ds(base + nfull * G, rem)])
        plsc.subcore_barrier()

        c0, nj = _group_range(wid, ng)
        pltpu.sync_copy(ei_hbm.at[1, pl.ds(c0, win)], idx_v)

        def edge_group(j, carry):
            pltpu.async_copy(ones_v, acc_sh.at[idx_v.at[j]], sem, add=True)
            return carry

        lax.fori_loop(0, nj, edge_group, 0)

        def edge_drain(j, carry):
            pltpu.make_async_copy(ones_v, acc_sh.at[idx_v.at[j]], sem).wait()
            return carry

        lax.fori_loop(0, nj, edge_drain, 0)
        plsc.subcore_barrier()

        lane = lax.iota(jnp.int32, LANES)
        zero16 = jnp.zeros((LANES,), jnp.int32)
        nchunks = nfull + (1 if rem else 0)
        for k in range(nchunks):
            off = base + k * G
            sz = G if k < nfull else rem
            pltpu.sync_copy(acc_sh.at[pl.ds(off, sz)],
                            buf_v if sz == G else buf_v.at[pl.ds(0, sz)])
            for j in range(_cdiv(sz, LANES)):
                rowi = jnp.minimum(j * LANES + lane, sz - 1)
                col_v[pl.ds(j * LANES, LANES)] = plsc.load_gather(
                    buf_v, [rowi, zero16])
            pltpu.sync_copy(col_v if sz == G else col_v.at[pl.ds(0, sz)],
                            out_hbm.at[cid, pl.ds(off, sz)])

    return deg_kernel


NBUF = 8


def _zero_acc(acc_sh, zbuf, sem, base, wr):
    nfz = wr // G
    rz = wr - nfz * G
    for k in range(nfz):
        pltpu.async_copy(zbuf, acc_sh.at[pl.ds(base + k * G, G)], sem)
    if rz:
        pltpu.async_copy(zbuf.at[pl.ds(0, rz)],
                         acc_sh.at[pl.ds(base + nfz * G, rz)], sem)
    for k in range(nfz):
        pltpu.make_async_copy(zbuf, acc_sh.at[pl.ds(base + k * G, G)],
                              sem).wait()
    if rz:
        pltpu.make_async_copy(zbuf.at[pl.ds(0, rz)],
                              acc_sh.at[pl.ds(base + nfz * G, rz)],
                              sem).wait()


def _edge_pipeline(y_src, idxs_v, idxd_v, rows_v, gsems, ssems, acc_sh, nj,
                   nbuf):
    def outer(g, carry):
        j0 = g * nbuf
        for b in range(nbuf):
            j = j0 + b

            @pl.when(jnp.logical_and(j < nj, j >= nbuf))
            def _(b=b, j=j):
                pltpu.make_async_copy(rows_v.at[b],
                                      acc_sh.at[idxd_v.at[j - nbuf]],
                                      ssems[b]).wait()

            @pl.when(j < nj)
            def _(b=b, j=j):
                pltpu.async_copy(y_src.at[idxs_v.at[j]], rows_v.at[b],
                                 gsems[b])
        for b in range(nbuf):
            j = j0 + b

            @pl.when(j < nj)
            def _(b=b, j=j):
                pltpu.make_async_copy(y_src.at[idxs_v.at[j]], rows_v.at[b],
                                      gsems[b]).wait()
                pltpu.async_copy(rows_v.at[b], acc_sh.at[idxd_v.at[j]],
                                 ssems[b], add=True)
        return carry

    lax.fori_loop(0, (nj + nbuf - 1) // nbuf, outer, 0)
    for b in range(nbuf):
        @pl.when(b < nj)
        def _(b=b):
            pltpu.make_async_copy(rows_v.at[b], acc_sh.at[idxd_v.at[0]],
                                  ssems[b]).wait()


def _copy_out(acc_sh, rows_v, ssems, out_hbm, cid, base, wr, nbuf):
    nfull = wr // G
    rem = wr - nfull * G
    nchunks = nfull + (1 if rem else 0)

    def chunk(k):
        if k < nfull:
            return base + k * G, G
        return base + nfull * G, rem

    def bufref(b, sz):
        return rows_v.at[b] if sz == G else rows_v.at[b, pl.ds(0, sz)]

    for k in range(nchunks):
        b = k % nbuf
        off, sz = chunk(k)
        if k >= nbuf:
            poff, psz = chunk(k - nbuf)
            pltpu.make_async_copy(bufref(b, psz),
                                  out_hbm.at[cid, pl.ds(poff, psz)],
                                  ssems[b]).wait()
        pltpu.sync_copy(acc_sh.at[pl.ds(off, sz)], bufref(b, sz))
        pltpu.async_copy(bufref(b, sz), out_hbm.at[cid, pl.ds(off, sz)],
                         ssems[b])
    for k in range(max(0, nchunks - nbuf), nchunks):
        b = k % nbuf
        off, sz = chunk(k)
        pltpu.make_async_copy(bufref(b, sz), out_hbm.at[cid, pl.ds(off, sz)],
                              ssems[b]).wait()


def _sc_agg(n, d, ng, nbuf=NBUF, dtype=jnp.float32):
    win = _win_size(ng)
    wr = _row_window(n)

    @functools.partial(
        pl.kernel,
        out_type=jax.ShapeDtypeStruct((NC, n, d), dtype),
        mesh=_mesh(),
        scratch_types=(
            [pltpu.VMEM((win, G), jnp.int32),
             pltpu.VMEM((win, G), jnp.int32),
             pltpu.VMEM((nbuf, G, d), dtype)]
            + [pltpu.SemaphoreType.DMA] * (2 * nbuf)
            + [pltpu.VMEM_SHARED((n, d), dtype)]
        ),
        compiler_params=pltpu.CompilerParams(use_tc_tiling_on_sc=False),
    )
    def agg_kernel(y_hbm, ei_hbm, z0_hbm, out_hbm,
                   idxs_v, idxd_v, rows_v, *rest):
        gsems = rest[:nbuf]
        ssems = rest[nbuf:2 * nbuf]
        acc_sh = rest[2 * nbuf]
        cid = lax.axis_index("c")
        sid = lax.axis_index("s")
        wid = sid * NC + cid

        pltpu.sync_copy(z0_hbm, rows_v.at[0])
        base = _row_start(sid, n)
        _zero_acc(acc_sh, rows_v.at[0], gsems[0], base, wr)
        plsc.subcore_barrier()

        c0, nj = _group_range(wid, ng)
        pltpu.sync_copy(ei_hbm.at[0, pl.ds(c0, win)], idxs_v)
        pltpu.sync_copy(ei_hbm.at[1, pl.ds(c0, win)], idxd_v)

        _edge_pipeline(y_hbm, idxs_v, idxd_v, rows_v,
                       gsems, ssems, acc_sh, nj, nbuf)

        plsc.subcore_barrier()
        _copy_out(acc_sh, rows_v, ssems, out_hbm, cid, base, wr, nbuf)

    return agg_kernel


TCBLK = 1000


def _dinv_from_partials(p_ref):
    deg = p_ref[0, 0] + p_ref[1, 0] + 1.0
    return jnp.transpose(lax.rsqrt(deg), (1, 0))


def _tc_scale(x, degp):
    n, din = x.shape
    blk = TCBLK

    def body(x_ref, p_ref, y_ref):
        dinv = _dinv_from_partials(p_ref)
        y_ref[...] = (x_ref[...] * dinv).astype(jnp.bfloat16)

    return pl.pallas_call(
        body,
        grid=(n // blk,),
        in_specs=[
            pl.BlockSpec((blk, din), lambda i: (i, 0)),
            pl.BlockSpec((NC, 1, 1, TCBLK), lambda i: (0, i, 0, 0)),
        ],
        out_specs=pl.BlockSpec((blk, din), lambda i: (i, 0)),
        out_shape=jax.ShapeDtypeStruct((n, din), jnp.bfloat16),
    )(x, degp)


def _tc_mlp(aggp, y, degp, W1, b1, W2):
    n, din = y.shape
    dhid = W1.shape[1]
    dout = W2.shape[1]
    blk = TCBLK

    def body(a_ref, y_ref, p_ref, w1_ref, b1_ref, w2_ref, z_ref):
        dinv = _dinv_from_partials(p_ref)
        s = (a_ref[0].astype(jnp.float32) + a_ref[1].astype(jnp.float32)
             + y_ref[...].astype(jnp.float32)) * dinv
        s = s.astype(jnp.bfloat16)
        h = jnp.dot(s, w1_ref[...], preferred_element_type=jnp.float32)
        h = jnp.maximum(h + b1_ref[...], 0.0).astype(jnp.bfloat16)
        z = jnp.dot(h, w2_ref[...], preferred_element_type=jnp.float32)
        z_ref[...] = (z * dinv).astype(jnp.bfloat16)

    return pl.pallas_call(
        body,
        grid=(n // blk,),
        in_specs=[
            pl.BlockSpec((NC, blk, din), lambda i: (0, i, 0)),
            pl.BlockSpec((blk, din), lambda i: (i, 0)),
            pl.BlockSpec((NC, 1, 1, TCBLK), lambda i: (0, i, 0, 0)),
            pl.BlockSpec((din, dhid), lambda i: (0, 0)),
            pl.BlockSpec((1, dhid), lambda i: (0, 0)),
            pl.BlockSpec((dhid, dout), lambda i: (0, 0)),
        ],
        out_specs=pl.BlockSpec((blk, dout), lambda i: (i, 0)),
        out_shape=jax.ShapeDtypeStruct((n, dout), jnp.bfloat16),
    )(aggp, y, degp, W1.astype(jnp.bfloat16), b1.reshape(1, dhid),
      W2.astype(jnp.bfloat16))


def _tc_final(cp, z, degp, b2):
    n, dout = z.shape
    blk = TCBLK

    def body(c_ref, z_ref, p_ref, b2_ref, o_ref):
        dinv = _dinv_from_partials(p_ref)
        o = (c_ref[0].astype(jnp.float32) + c_ref[1].astype(jnp.float32)
             + z_ref[...].astype(jnp.float32)) * dinv + b2_ref[...]
        m = jnp.max(o, axis=1, keepdims=True)
        e = jnp.exp(o - m)
        s = jnp.sum(e, axis=1, keepdims=True)
        o_ref[...] = (o - m) - jnp.log(s)

    return pl.pallas_call(
        body,
        grid=(n // blk,),
        in_specs=[
            pl.BlockSpec((NC, blk, dout), lambda i: (0, i, 0)),
            pl.BlockSpec((blk, dout), lambda i: (i, 0)),
            pl.BlockSpec((NC, 1, 1, TCBLK), lambda i: (0, i, 0, 0)),
            pl.BlockSpec((1, dout), lambda i: (0, 0)),
        ],
        out_specs=pl.BlockSpec((blk, dout), lambda i: (i, 0)),
        out_shape=jax.ShapeDtypeStruct((n, dout), jnp.float32),
    )(cp, z, degp, b2.reshape(1, dout))


def kernel(x, edge_index, W1, b1, W2, b2):
    n, din = x.shape
    e = edge_index.shape[1]
    ng = e // G
    ei3 = edge_index.reshape(2, ng, G)

    degp = _sc_degree(n, ng)(ei3)
    degp4 = degp.reshape(NC, n // TCBLK, 1, TCBLK)
    y = _tc_scale(x, degp4)
    aggp = _sc_agg(n, din, ng, nbuf=6, dtype=jnp.bfloat16)(
        y, ei3, jnp.zeros((G, din), jnp.bfloat16))
    z = _tc_mlp(aggp, y, degp4, W1, b1, W2)
    cp = _sc_agg(n, W2.shape[1], ng, dtype=jnp.bfloat16)(
        z, ei3, jnp.zeros((G, 16), jnp.bfloat16))
    return _tc_final(cp, z, degp4, b2)


# speedup vs baseline: 58.0824x; 1.0030x over previous
import functools

import jax
import jax.numpy as jnp
from jax import lax
from jax.experimental import pallas as pl
from jax.experimental.pallas import tpu as pltpu
from jax.experimental.pallas import tpu_sc as plsc

NC = 2
NS = 16
NW = NC * NS
LANES = 16
G = 128


def _mesh():
    return plsc.VectorSubcoreMesh(core_axis_name="c", subcore_axis_name="s")


def _cdiv(a, b):
    return (a + b - 1) // b


def _win_size(ng):
    nb = ng // 8
    return 8 * _cdiv(nb, NW) + (ng - 8 * nb)


def _group_range(wid, ng):
    nb = ng // 8
    rg = ng - 8 * nb
    b0 = (nb * wid) // NW
    b1 = (nb * (wid + 1)) // NW
    c0 = 8 * b0
    nj = 8 * (b1 - b0) + jnp.where(wid == NW - 1, rg, 0)
    return c0, nj


def _row_window(n):
    return 8 * _cdiv(n // 8, NS)


def _row_start(sid, n):
    return 8 * (((n // 8) * sid) // NS)


def _sc_degree(n, ng):
    win = _win_size(ng)
    wr = _row_window(n)

    @functools.partial(
        pl.kernel,
        out_type=jax.ShapeDtypeStruct((NC, n), jnp.float32),
        mesh=_mesh(),
        scratch_types=[
            pltpu.VMEM((win, G), jnp.int32),
            pltpu.VMEM((G, LANES), jnp.float32),
            pltpu.VMEM((G, LANES), jnp.float32),
            pltpu.VMEM((G,), jnp.float32),
            pltpu.SemaphoreType.DMA,
            pltpu.VMEM_SHARED((n, LANES), jnp.float32),
        ],
        compiler_params=pltpu.CompilerParams(use_tc_tiling_on_sc=False,
                                             needs_layout_passes=False),
    )
    def deg_kernel(ei_hbm, out_hbm, idx_v, ones_v, buf_v, col_v, sem, acc_sh):
        cid = lax.axis_index("c")
        sid = lax.axis_index("s")
        wid = sid * NC + cid

        def initrow(i, carry):
            ones_v[i, :] = jnp.full((LANES,), 1.0, jnp.float32)
            buf_v[i, :] = jnp.zeros((LANES,), jnp.float32)
            return carry

        lax.fori_loop(0, G, initrow, 0)

        base = _row_start(sid, n)
        nfull = wr // G
        rem = wr - nfull * G
        for k in range(nfull):
            pltpu.sync_copy(buf_v, acc_sh.at[pl.ds(base + k * G, G)])
        if rem:
            pltpu.sync_copy(buf_v.at[pl.ds(0, rem)],
                            acc_sh.at[pl.ds(base + nfull * G, rem)])
        plsc.subcore_barrier()

        c0, nj = _group_range(wid, ng)
        pltpu.sync_copy(ei_hbm.at[1, pl.ds(c0, win)], idx_v)

        def edge_group(j, carry):
            pltpu.async_copy(ones_v, acc_sh.at[idx_v.at[j]], sem, add=True)
            return carry

        lax.fori_loop(0, nj, edge_group, 0)

        def edge_drain(j, carry):
            pltpu.make_async_copy(ones_v, acc_sh.at[idx_v.at[j]], sem).wait()
            return carry

        lax.fori_loop(0, nj, edge_drain, 0)
        plsc.subcore_barrier()

        lane = lax.iota(jnp.int32, LANES)
        zero16 = jnp.zeros((LANES,), jnp.int32)
        nchunks = nfull + (1 if rem else 0)
        for k in range(nchunks):
            off = base + k * G
            sz = G if k < nfull else rem
            pltpu.sync_copy(acc_sh.at[pl.ds(off, sz)],
                            buf_v if sz == G else buf_v.at[pl.ds(0, sz)])
            for j in range(_cdiv(sz, LANES)):
                rowi = jnp.minimum(j * LANES + lane, sz - 1)
                col_v[pl.ds(j * LANES, LANES)] = plsc.load_gather(
                    buf_v, [rowi, zero16])
            pltpu.sync_copy(col_v if sz == G else col_v.at[pl.ds(0, sz)],
                            out_hbm.at[cid, pl.ds(off, sz)])

    return deg_kernel


NBUF = 8


def _zero_acc(acc_sh, zbuf, sem, base, wr):
    nfz = wr // G
    rz = wr - nfz * G
    for k in range(nfz):
        pltpu.async_copy(zbuf, acc_sh.at[pl.ds(base + k * G, G)], sem)
    if rz:
        pltpu.async_copy(zbuf.at[pl.ds(0, rz)],
                         acc_sh.at[pl.ds(base + nfz * G, rz)], sem)
    for k in range(nfz):
        pltpu.make_async_copy(zbuf, acc_sh.at[pl.ds(base + k * G, G)],
                              sem).wait()
    if rz:
        pltpu.make_async_copy(zbuf.at[pl.ds(0, rz)],
                              acc_sh.at[pl.ds(base + nfz * G, rz)],
                              sem).wait()


def _edge_pipeline(y_src, idxs_v, idxd_v, rows_v, gsems, ssems, acc_sh, nj,
                   nbuf):
    def outer(g, carry):
        j0 = g * nbuf
        for b in range(nbuf):
            j = j0 + b

            @pl.when(jnp.logical_and(j < nj, j >= nbuf))
            def _(b=b, j=j):
                pltpu.make_async_copy(rows_v.at[b],
                                      acc_sh.at[idxd_v.at[j - nbuf]],
                                      ssems[b]).wait()

            @pl.when(j < nj)
            def _(b=b, j=j):
                pltpu.async_copy(y_src.at[idxs_v.at[j]], rows_v.at[b],
                                 gsems[b])
        for b in range(nbuf):
            j = j0 + b

            @pl.when(j < nj)
            def _(b=b, j=j):
                pltpu.make_async_copy(y_src.at[idxs_v.at[j]], rows_v.at[b],
                                      gsems[b]).wait()
                pltpu.async_copy(rows_v.at[b], acc_sh.at[idxd_v.at[j]],
                                 ssems[b], add=True)
        return carry

    lax.fori_loop(0, (nj + nbuf - 1) // nbuf, outer, 0)
    for b in range(nbuf):
        @pl.when(b < nj)
        def _(b=b):
            pltpu.make_async_copy(rows_v.at[b], acc_sh.at[idxd_v.at[0]],
                                  ssems[b]).wait()


def _copy_out(acc_sh, rows_v, ssems, out_hbm, cid, base, wr, nbuf):
    nfull = wr // G
    rem = wr - nfull * G
    nchunks = nfull + (1 if rem else 0)

    def chunk(k):
        if k < nfull:
            return base + k * G, G
        return base + nfull * G, rem

    def bufref(b, sz):
        return rows_v.at[b] if sz == G else rows_v.at[b, pl.ds(0, sz)]

    for k in range(nchunks):
        b = k % nbuf
        off, sz = chunk(k)
        if k >= nbuf:
            poff, psz = chunk(k - nbuf)
            pltpu.make_async_copy(bufref(b, psz),
                                  out_hbm.at[cid, pl.ds(poff, psz)],
                                  ssems[b]).wait()
        pltpu.sync_copy(acc_sh.at[pl.ds(off, sz)], bufref(b, sz))
        pltpu.async_copy(bufref(b, sz), out_hbm.at[cid, pl.ds(off, sz)],
                         ssems[b])
    for k in range(max(0, nchunks - nbuf), nchunks):
        b = k % nbuf
        off, sz = chunk(k)
        pltpu.make_async_copy(bufref(b, sz), out_hbm.at[cid, pl.ds(off, sz)],
                              ssems[b]).wait()


def _sc_agg(n, d, ng, nbuf=NBUF, dtype=jnp.float32):
    win = _win_size(ng)
    wr = _row_window(n)

    @functools.partial(
        pl.kernel,
        out_type=jax.ShapeDtypeStruct((NC, n, d), dtype),
        mesh=_mesh(),
        scratch_types=(
            [pltpu.VMEM((win, G), jnp.int32),
             pltpu.VMEM((win, G), jnp.int32),
             pltpu.VMEM((nbuf, G, d), dtype)]
            + [pltpu.SemaphoreType.DMA] * (2 * nbuf)
            + [pltpu.VMEM_SHARED((n, d), dtype)]
        ),
        compiler_params=pltpu.CompilerParams(use_tc_tiling_on_sc=False),
    )
    def agg_kernel(y_hbm, ei_hbm, z0_hbm, out_hbm,
                   idxs_v, idxd_v, rows_v, *rest):
        gsems = rest[:nbuf]
        ssems = rest[nbuf:2 * nbuf]
        acc_sh = rest[2 * nbuf]
        cid = lax.axis_index("c")
        sid = lax.axis_index("s")
        wid = sid * NC + cid

        pltpu.sync_copy(z0_hbm, rows_v.at[0])
        base = _row_start(sid, n)
        _zero_acc(acc_sh, rows_v.at[0], gsems[0], base, wr)
        plsc.subcore_barrier()

        c0, nj = _group_range(wid, ng)
        pltpu.sync_copy(ei_hbm.at[0, pl.ds(c0, win)], idxs_v)
        pltpu.sync_copy(ei_hbm.at[1, pl.ds(c0, win)], idxd_v)

        _edge_pipeline(y_hbm, idxs_v, idxd_v, rows_v,
                       gsems, ssems, acc_sh, nj, nbuf)

        plsc.subcore_barrier()
        _copy_out(acc_sh, rows_v, ssems, out_hbm, cid, base, wr, nbuf)

    return agg_kernel


TCBLK = 1000


def _dinv_from_partials(p_ref):
    deg = p_ref[0, 0] + p_ref[1, 0] + 1.0
    return jnp.transpose(lax.rsqrt(deg), (1, 0))


def _tc_scale(x, degp):
    n, din = x.shape
    blk = TCBLK

    def body(x_ref, p_ref, y_ref):
        dinv = _dinv_from_partials(p_ref)
        y_ref[...] = (x_ref[...] * dinv).astype(jnp.bfloat16)

    return pl.pallas_call(
        body,
        grid=(n // blk,),
        in_specs=[
            pl.BlockSpec((blk, din), lambda i: (i, 0)),
            pl.BlockSpec((NC, 1, 1, TCBLK), lambda i: (0, i, 0, 0)),
        ],
        out_specs=pl.BlockSpec((blk, din), lambda i: (i, 0)),
        out_shape=jax.ShapeDtypeStruct((n, din), jnp.bfloat16),
    )(x, degp)


def _tc_mlp(aggp, y, degp, W1, b1, W2):
    n, din = y.shape
    dhid = W1.shape[1]
    dout = W2.shape[1]
    blk = TCBLK

    def body(a_ref, y_ref, p_ref, w1_ref, b1_ref, w2_ref, z_ref):
        dinv = _dinv_from_partials(p_ref)
        s = (a_ref[0].astype(jnp.float32) + a_ref[1].astype(jnp.float32)
             + y_ref[...].astype(jnp.float32)) * dinv
        s = s.astype(jnp.bfloat16)
        h = jnp.dot(s, w1_ref[...], preferred_element_type=jnp.float32)
        h = jnp.maximum(h + b1_ref[...], 0.0).astype(jnp.bfloat16)
        z = jnp.dot(h, w2_ref[...], preferred_element_type=jnp.float32)
        z_ref[...] = (z * dinv).astype(jnp.bfloat16)

    return pl.pallas_call(
        body,
        grid=(n // blk,),
        in_specs=[
            pl.BlockSpec((NC, blk, din), lambda i: (0, i, 0)),
            pl.BlockSpec((blk, din), lambda i: (i, 0)),
            pl.BlockSpec((NC, 1, 1, TCBLK), lambda i: (0, i, 0, 0)),
            pl.BlockSpec((din, dhid), lambda i: (0, 0)),
            pl.BlockSpec((1, dhid), lambda i: (0, 0)),
            pl.BlockSpec((dhid, dout), lambda i: (0, 0)),
        ],
        out_specs=pl.BlockSpec((blk, dout), lambda i: (i, 0)),
        out_shape=jax.ShapeDtypeStruct((n, dout), jnp.bfloat16),
    )(aggp, y, degp, W1.astype(jnp.bfloat16), b1.reshape(1, dhid),
      W2.astype(jnp.bfloat16))


def _tc_final(cp, z, degp, b2):
    n, dout = z.shape
    blk = TCBLK

    def body(c_ref, z_ref, p_ref, b2_ref, o_ref):
        dinv = _dinv_from_partials(p_ref)
        o = (c_ref[0].astype(jnp.float32) + c_ref[1].astype(jnp.float32)
             + z_ref[...].astype(jnp.float32)) * dinv + b2_ref[...]
        m = jnp.max(o, axis=1, keepdims=True)
        e = jnp.exp(o - m)
        s = jnp.sum(e, axis=1, keepdims=True)
        o_ref[...] = (o - m) - jnp.log(s)

    return pl.pallas_call(
        body,
        grid=(n // blk,),
        in_specs=[
            pl.BlockSpec((NC, blk, dout), lambda i: (0, i, 0)),
            pl.BlockSpec((blk, dout), lambda i: (i, 0)),
            pl.BlockSpec((NC, 1, 1, TCBLK), lambda i: (0, i, 0, 0)),
            pl.BlockSpec((1, dout), lambda i: (0, 0)),
        ],
        out_specs=pl.BlockSpec((blk, dout), lambda i: (i, 0)),
        out_shape=jax.ShapeDtypeStruct((n, dout), jnp.float32),
    )(cp, z, degp, b2.reshape(1, dout))


def kernel(x, edge_index, W1, b1, W2, b2):
    n, din = x.shape
    e = edge_index.shape[1]
    ng = e // G
    ei3 = edge_index.reshape(2, ng, G)

    degp = _sc_degree(n, ng)(ei3)
    degp4 = degp.reshape(NC, n // TCBLK, 1, TCBLK)
    y = _tc_scale(x, degp4)
    aggp = _sc_agg(n, din, ng, nbuf=8, dtype=jnp.bfloat16)(
        y, ei3, jnp.zeros((G, din), jnp.bfloat16))
    z = _tc_mlp(aggp, y, degp4, W1, b1, W2)
    cp = _sc_agg(n, W2.shape[1], ng, dtype=jnp.bfloat16)(
        z, ei3, jnp.zeros((G, 16), jnp.bfloat16))
    return _tc_final(cp, z, degp4, b2)
